# baseline (speedup 1.0000x reference)
import functools

import jax
import jax.numpy as jnp
import numpy as np
from jax import lax
from jax.experimental import pallas as pl
from jax.experimental.pallas import tpu as pltpu
from jax.experimental.pallas import tpu_sc as plsc

_MAGIC = np.float32(12582912.0)
_KMAGIC = 1262485504
L = 16


def _mesh():
    return plsc.VectorSubcoreMesh(core_axis_name="c", subcore_axis_name="s")


def _wid():
    info = plsc.get_sparse_core_info()
    return lax.axis_index("s") * info.num_cores + lax.axis_index("c")


def _make_pass_a(R, C, NW):
    rows_w = R // NW
    nblk = rows_w // L

    @functools.partial(
        pl.kernel,
        out_type=(
            jax.ShapeDtypeStruct((R,), jnp.float32),
            jax.ShapeDtypeStruct((NW * L,), jnp.float32),
        ),
        mesh=_mesh(),
        compiler_params=pltpu.CompilerParams(needs_layout_passes=False),
        scratch_types=[
            pltpu.VMEM((L * C,), jnp.float32),
            pltpu.VMEM((rows_w,), jnp.float32),
            pltpu.VMEM((L,), jnp.float32),
        ],
    )
    def pass_a(x_hbm, rowmax_hbm, gpart_hbm, xb, rmv, gv):
        wid = _wid()
        base = wid * rows_w
        ivec = lax.iota(jnp.int32, L) * C

        def blk_body(b, g):
            pltpu.sync_copy(x_hbm.at[pl.ds((base + b * L) * C, L * C)], xb)

            def col_body(col, carry):
                v = plsc.load_gather(xb, [ivec + col])
                return (jnp.maximum(carry[0], v), jnp.minimum(carry[1], v))

            mxv, mnv = lax.fori_loop(
                0, C, col_body,
                (jnp.full((L,), -jnp.inf, jnp.float32),
                 jnp.full((L,), jnp.inf, jnp.float32)),
            )
            rmv[pl.ds(b * L, L)] = mxv
            return jnp.minimum(g, mnv - mxv)

        g = lax.fori_loop(
            0, nblk, blk_body, jnp.full((L,), jnp.inf, jnp.float32))
        gv[...] = g
        pltpu.sync_copy(rmv, rowmax_hbm.at[pl.ds(base, rows_w)])
        pltpu.sync_copy(gv, gpart_hbm.at[pl.ds(wid * L, L)])

    return pass_a


def _make_pass_b(R, C, NW):
    rows_w = R // NW
    nblk = rows_w // L

    @functools.partial(
        pl.kernel,
        out_type=(
            jax.ShapeDtypeStruct((R,), jnp.float32),
            jax.ShapeDtypeStruct((R,), jnp.float32),
        ),
        mesh=_mesh(),
        compiler_params=pltpu.CompilerParams(needs_layout_passes=False),
        scratch_types=[
            pltpu.VMEM((L * C,), jnp.float32),
            pltpu.VMEM((rows_w,), jnp.float32),
            pltpu.VMEM((rows_w,), jnp.float32),
            pltpu.VMEM((rows_w,), jnp.float32),
            pltpu.VMEM((256,), jnp.float32),
            pltpu.VMEM((2 * L,), jnp.float32),
        ],
    )
    def pass_b(x_hbm, rowmax_hbm, lut_hbm, scal_hbm, rowsum_hbm, emin_hbm,
               xb, rmv, rsv, emv, lutv, scv):
        wid = _wid()
        base = wid * rows_w
        ivec = lax.iota(jnp.int32, L) * C
        pltpu.sync_copy(lut_hbm, lutv)
        pltpu.sync_copy(scal_hbm, scv)
        pltpu.sync_copy(rowmax_hbm.at[pl.ds(base, rows_w)], rmv)
        s1 = scv[pl.ds(0, L)]
        zp1 = scv[pl.ds(L, L)]

        def blk_body(b, _):
            pltpu.sync_copy(x_hbm.at[pl.ds((base + b * L) * C, L * C)], xb)
            rm = rmv[pl.ds(b * L, L)]

            def col_body(col, carry):
                v = plsc.load_gather(xb, [ivec + col])
                t = (v - rm) / s1 + zp1
                iu = lax.bitcast_convert_type(t + _MAGIC, jnp.int32)
                idx = jnp.clip(iu - (_KMAGIC - 128), 0, 255)
                e = plsc.load_gather(lutv, [idx])
                return (carry[0] + e, jnp.minimum(carry[1], e))

            sacc, macc = lax.fori_loop(
                0, C, col_body,
                (jnp.zeros((L,), jnp.float32),
                 jnp.full((L,), jnp.inf, jnp.float32)),
            )
            rsv[pl.ds(b * L, L)] = sacc
            emv[pl.ds(b * L, L)] = macc
            return 0

        lax.fori_loop(0, nblk, blk_body, 0)
        pltpu.sync_copy(rsv, rowsum_hbm.at[pl.ds(base, rows_w)])
        pltpu.sync_copy(emv, emin_hbm.at[pl.ds(base, rows_w)])

    return pass_b


def _make_pass_c(R, C, NW):
    rows_w = R // NW
    nblk = rows_w // L

    @functools.partial(
        pl.kernel,
        out_type=jax.ShapeDtypeStruct((R * C,), jnp.float32),
        mesh=_mesh(),
        compiler_params=pltpu.CompilerParams(needs_layout_passes=False),
        scratch_types=[
            pltpu.VMEM((L * C,), jnp.float32),
            pltpu.VMEM((L * C,), jnp.float32),
            pltpu.VMEM((rows_w,), jnp.float32),
            pltpu.VMEM((rows_w,), jnp.float32),
            pltpu.VMEM((256,), jnp.float32),
            pltpu.VMEM((4 * L,), jnp.float32),
        ],
    )
    def pass_c(x_hbm, rowmax_hbm, rowsum_hbm, lut_hbm, scal_hbm, out_hbm,
               xb, ob, rmv, rsv, lutv, scv):
        wid = _wid()
        base = wid * rows_w
        ivec = lax.iota(jnp.int32, L) * C
        pltpu.sync_copy(lut_hbm, lutv)
        pltpu.sync_copy(scal_hbm, scv)
        pltpu.sync_copy(rowmax_hbm.at[pl.ds(base, rows_w)], rmv)
        pltpu.sync_copy(rowsum_hbm.at[pl.ds(base, rows_w)], rsv)
        s1 = scv[pl.ds(0, L)]
        zp1 = scv[pl.ds(L, L)]
        s2 = scv[pl.ds(2 * L, L)]
        zp2 = scv[pl.ds(3 * L, L)]

        def blk_body(b, _):
            pltpu.sync_copy(x_hbm.at[pl.ds((base + b * L) * C, L * C)], xb)
            rm = rmv[pl.ds(b * L, L)]
            rs = rsv[pl.ds(b * L, L)]

            def col_body(col, _):
                v = plsc.load_gather(xb, [ivec + col])
                t = (v - rm) / s1 + zp1
                iu = lax.bitcast_convert_type(t + _MAGIC, jnp.int32)
                idx = jnp.clip(iu - (_KMAGIC - 128), 0, 255)
                e = plsc.load_gather(lutv, [idx])
                o = e / rs
                t2 = o / s2 + zp2
                r2 = (t2 + _MAGIC) - _MAGIC
                q2 = jnp.clip(r2, -128.0, 127.0)
                plsc.store_scatter(ob, [ivec + col], (q2 - zp2) * s2)
                return 0

            lax.fori_loop(0, C, col_body, 0)
            pltpu.sync_copy(ob, out_hbm.at[pl.ds((base + b * L) * C, L * C)])
            return 0

        lax.fori_loop(0, nblk, blk_body, 0)

    return pass_c


def kernel(inputs):
    B, H, S, C = inputs.shape
    R = B * H * S
    NW = 32
    x = inputs.reshape(R * C)

    rowmax, gpart = _make_pass_a(R, C, NW)(x)
    mn = jnp.min(gpart)
    mx = jnp.float32(0.0)
    scale1 = jnp.maximum((mx - mn) / 255.0, jnp.float32(1e-12))
    zp1 = jnp.clip(jnp.round(-128.0 - mn / scale1), -128, 127)
    lut = jnp.exp(((jnp.arange(256, dtype=jnp.float32) - 128.0) - zp1) * scale1)
    scal1 = jnp.concatenate(
        [jnp.full((L,), scale1), jnp.full((L,), zp1)]).astype(jnp.float32)

    rowsum, emin = _make_pass_b(R, C, NW)(x, rowmax, lut, scal1)
    outmn = jnp.min(emin / rowsum)
    outmx = jnp.max(1.0 / rowsum)
    scale2 = jnp.maximum((outmx - outmn) / 255.0, jnp.float32(1e-12))
    zp2 = jnp.clip(jnp.round(-128.0 - outmn / scale2), -128, 127)
    scal2 = jnp.concatenate(
        [jnp.full((L,), scale1), jnp.full((L,), zp1),
         jnp.full((L,), scale2), jnp.full((L,), zp2)]).astype(jnp.float32)

    out = _make_pass_c(R, C, NW)(x, rowmax, rowsum, lut, scal2)
    return out.reshape(inputs.shape)


# speedup vs baseline: 1.1778x; 1.1778x over previous
import functools

import jax
import jax.numpy as jnp
import numpy as np
from jax import lax
from jax.experimental import pallas as pl
from jax.experimental.pallas import tpu as pltpu
from jax.experimental.pallas import tpu_sc as plsc

_MAGIC = np.float32(12582912.0)
_KMAGIC = 1262485504
L = 16
U = 8


def _mesh():
    return plsc.VectorSubcoreMesh(core_axis_name="c", subcore_axis_name="s")


def _wid():
    info = plsc.get_sparse_core_info()
    return lax.axis_index("s") * info.num_cores + lax.axis_index("c")


def _tree(fn, vs):
    vs = list(vs)
    while len(vs) > 1:
        vs = [fn(vs[i], vs[i + 1]) for i in range(0, len(vs) - 1, 2)] + (
            [vs[-1]] if len(vs) % 2 else [])
    return vs[0]


def _quant_idx(v, inv1, off):
    t = v * inv1 + off
    iu = lax.bitcast_convert_type(t + _MAGIC, jnp.int32)
    return jnp.clip(iu - (_KMAGIC - 128), 0, 255)


def _make_pass_a(R, C, NW):
    rows_w = R // NW
    nblk = rows_w // L

    @functools.partial(
        pl.kernel,
        out_type=(
            jax.ShapeDtypeStruct((R,), jnp.float32),
            jax.ShapeDtypeStruct((NW * L,), jnp.float32),
        ),
        mesh=_mesh(),
        compiler_params=pltpu.CompilerParams(needs_layout_passes=False),
        scratch_types=[
            pltpu.VMEM((L * C,), jnp.float32),
            pltpu.VMEM((rows_w,), jnp.float32),
            pltpu.VMEM((L,), jnp.float32),
        ],
    )
    def pass_a(x_hbm, rowmax_hbm, gpart_hbm, xb, rmv, gv):
        wid = _wid()
        base = wid * rows_w
        ivec = lax.iota(jnp.int32, L) * C

        def blk_body(b, g):
            pltpu.sync_copy(x_hbm.at[pl.ds((base + b * L) * C, L * C)], xb)

            def col_body(cb, carry):
                mx, mn = carry
                bi = ivec + cb * U
                vs = [plsc.load_gather(xb, [bi + u]) for u in range(U)]
                mx = jnp.maximum(mx, _tree(jnp.maximum, vs))
                mn = jnp.minimum(mn, _tree(jnp.minimum, vs))
                return (mx, mn)

            mxv, mnv = lax.fori_loop(
                0, C // U, col_body,
                (jnp.full((L,), -jnp.inf, jnp.float32),
                 jnp.full((L,), jnp.inf, jnp.float32)),
            )
            rmv[pl.ds(b * L, L)] = mxv
            return jnp.minimum(g, mnv - mxv)

        g = lax.fori_loop(
            0, nblk, blk_body, jnp.full((L,), jnp.inf, jnp.float32))
        gv[...] = g
        pltpu.sync_copy(rmv, rowmax_hbm.at[pl.ds(base, rows_w)])
        pltpu.sync_copy(gv, gpart_hbm.at[pl.ds(wid * L, L)])

    return pass_a


def _make_pass_b(R, C, NW):
    rows_w = R // NW
    nblk = rows_w // L

    @functools.partial(
        pl.kernel,
        out_type=(
            jax.ShapeDtypeStruct((R,), jnp.float32),
            jax.ShapeDtypeStruct((R,), jnp.float32),
        ),
        mesh=_mesh(),
        compiler_params=pltpu.CompilerParams(needs_layout_passes=False),
        scratch_types=[
            pltpu.VMEM((L * C,), jnp.float32),
            pltpu.VMEM((rows_w,), jnp.float32),
            pltpu.VMEM((rows_w,), jnp.float32),
            pltpu.VMEM((rows_w,), jnp.float32),
            pltpu.VMEM((256,), jnp.float32),
            pltpu.VMEM((2 * L,), jnp.float32),
        ],
    )
    def pass_b(x_hbm, rowmax_hbm, lut_hbm, scal_hbm, rowsum_hbm, emin_hbm,
               xb, rmv, rsv, emv, lutv, scv):
        wid = _wid()
        base = wid * rows_w
        ivec = lax.iota(jnp.int32, L) * C
        pltpu.sync_copy(lut_hbm, lutv)
        pltpu.sync_copy(scal_hbm, scv)
        pltpu.sync_copy(rowmax_hbm.at[pl.ds(base, rows_w)], rmv)
        inv1 = scv[pl.ds(0, L)]
        zp1 = scv[pl.ds(L, L)]

        def blk_body(b, _):
            pltpu.sync_copy(x_hbm.at[pl.ds((base + b * L) * C, L * C)], xb)
            rm = rmv[pl.ds(b * L, L)]
            off = zp1 - rm * inv1

            def col_body(cb, carry):
                sacc, macc = carry
                bi = ivec + cb * U
                es = []
                for u in range(U):
                    v = plsc.load_gather(xb, [bi + u])
                    idx = _quant_idx(v, inv1, off)
                    es.append(plsc.load_gather(lutv, [idx]))
                sacc = sacc + _tree(jnp.add, es)
                macc = jnp.minimum(macc, _tree(jnp.minimum, es))
                return (sacc, macc)

            sacc, macc = lax.fori_loop(
                0, C // U, col_body,
                (jnp.zeros((L,), jnp.float32),
                 jnp.full((L,), jnp.inf, jnp.float32)),
            )
            rsv[pl.ds(b * L, L)] = sacc
            emv[pl.ds(b * L, L)] = macc
            return 0

        lax.fori_loop(0, nblk, blk_body, 0)
        pltpu.sync_copy(rsv, rowsum_hbm.at[pl.ds(base, rows_w)])
        pltpu.sync_copy(emv, emin_hbm.at[pl.ds(base, rows_w)])

    return pass_b


def _make_pass_c(R, C, NW):
    rows_w = R // NW
    nblk = rows_w // L

    @functools.partial(
        pl.kernel,
        out_type=jax.ShapeDtypeStruct((R * C,), jnp.float32),
        mesh=_mesh(),
        compiler_params=pltpu.CompilerParams(needs_layout_passes=False),
        scratch_types=[
            pltpu.VMEM((L * C,), jnp.float32),
            pltpu.VMEM((L * C,), jnp.float32),
            pltpu.VMEM((rows_w,), jnp.float32),
            pltpu.VMEM((rows_w,), jnp.float32),
            pltpu.VMEM((256,), jnp.float32),
            pltpu.VMEM((5 * L,), jnp.float32),
        ],
    )
    def pass_c(x_hbm, rowmax_hbm, rowsum_hbm, lut_hbm, scal_hbm, out_hbm,
               xb, ob, rmv, rsv, lutv, scv):
        wid = _wid()
        base = wid * rows_w
        ivec = lax.iota(jnp.int32, L) * C
        pltpu.sync_copy(lut_hbm, lutv)
        pltpu.sync_copy(scal_hbm, scv)
        pltpu.sync_copy(rowmax_hbm.at[pl.ds(base, rows_w)], rmv)
        pltpu.sync_copy(rowsum_hbm.at[pl.ds(base, rows_w)], rsv)
        inv1 = scv[pl.ds(0, L)]
        zp1 = scv[pl.ds(L, L)]
        inv2 = scv[pl.ds(2 * L, L)]
        zp2 = scv[pl.ds(3 * L, L)]
        s2 = scv[pl.ds(4 * L, L)]

        def blk_body(b, _):
            pltpu.sync_copy(x_hbm.at[pl.ds((base + b * L) * C, L * C)], xb)
            rm = rmv[pl.ds(b * L, L)]
            rs = rsv[pl.ds(b * L, L)]
            off = zp1 - rm * inv1
            k2 = (1.0 / rs) * inv2

            def col_body(cb, _):
                bi = ivec + cb * U
                for u in range(U):
                    v = plsc.load_gather(xb, [bi + u])
                    idx = _quant_idx(v, inv1, off)
                    e = plsc.load_gather(lutv, [idx])
                    t2 = e * k2 + zp2
                    r2 = (t2 + _MAGIC) - _MAGIC
                    q2 = jnp.clip(r2, -128.0, 127.0)
                    plsc.store_scatter(ob, [bi + u], (q2 - zp2) * s2)
                return 0

            lax.fori_loop(0, C // U, col_body, 0)
            pltpu.sync_copy(ob, out_hbm.at[pl.ds((base + b * L) * C, L * C)])
            return 0

        lax.fori_loop(0, nblk, blk_body, 0)

    return pass_c


def kernel(inputs):
    B, H, S, C = inputs.shape
    R = B * H * S
    NW = 32
    x = inputs.reshape(R * C)

    rowmax, gpart = _make_pass_a(R, C, NW)(x)
    mn = jnp.min(gpart)
    mx = jnp.float32(0.0)
    scale1 = jnp.maximum((mx - mn) / 255.0, jnp.float32(1e-12))
    zp1 = jnp.clip(jnp.round(-128.0 - mn / scale1), -128, 127)
    lut = jnp.exp(((jnp.arange(256, dtype=jnp.float32) - 128.0) - zp1) * scale1)
    inv1 = 1.0 / scale1
    scal1 = jnp.concatenate(
        [jnp.full((L,), inv1), jnp.full((L,), zp1)]).astype(jnp.float32)

    rowsum, emin = _make_pass_b(R, C, NW)(x, rowmax, lut, scal1)
    outmn = jnp.min(emin / rowsum)
    outmx = jnp.max(1.0 / rowsum)
    scale2 = jnp.maximum((outmx - outmn) / 255.0, jnp.float32(1e-12))
    zp2 = jnp.clip(jnp.round(-128.0 - outmn / scale2), -128, 127)
    scal2 = jnp.concatenate(
        [jnp.full((L,), inv1), jnp.full((L,), zp1),
         jnp.full((L,), 1.0 / scale2), jnp.full((L,), zp2),
         jnp.full((L,), scale2)]).astype(jnp.float32)

    out = _make_pass_c(R, C, NW)(x, rowmax, rowsum, lut, scal2)
    return out.reshape(inputs.shape)


# speedup vs baseline: 2.9678x; 2.5198x over previous
import functools

import jax
import jax.numpy as jnp
import numpy as np
from jax import lax
from jax.experimental import pallas as pl
from jax.experimental.pallas import tpu as pltpu
from jax.experimental.pallas import tpu_sc as plsc

_MAGIC = np.float32(12582912.0)
_KMAGIC = 1262485504
_KME = _KMAGIC - 384
L = 16
U = 4


def _mesh():
    return plsc.VectorSubcoreMesh(core_axis_name="c", subcore_axis_name="s")


def _wid():
    info = plsc.get_sparse_core_info()
    return lax.axis_index("s") * info.num_cores + lax.axis_index("c")


def _tree(fn, vs):
    vs = list(vs)
    while len(vs) > 1:
        vs = [fn(vs[i], vs[i + 1]) for i in range(0, len(vs) - 1, 2)] + (
            [vs[-1]] if len(vs) % 2 else [])
    return vs[0]


def _lane_iota():
    return lax.iota(jnp.int32, L)


def _code16(v, inv1, off, lane):
    t = v * inv1 + off
    iu = lax.bitcast_convert_type(t + _MAGIC, jnp.int32)
    return jnp.left_shift(iu - _KME, 4) + lane


def _splat(ref, i):
    return plsc.load_gather(ref, [jnp.full((L,), 0, jnp.int32) + i])


def _make_pass_a(R, C, NW, BLKR):
    rows_w = R // NW
    nblk = rows_w // BLKR

    @functools.partial(
        pl.kernel,
        out_type=(
            jax.ShapeDtypeStruct((R,), jnp.float32),
            jax.ShapeDtypeStruct((NW * L,), jnp.float32),
        ),
        mesh=_mesh(),
        compiler_params=pltpu.CompilerParams(needs_layout_passes=False),
        scratch_types=[
            pltpu.VMEM((BLKR * C,), jnp.float32),
            pltpu.VMEM((rows_w,), jnp.float32),
            pltpu.VMEM((L,), jnp.float32),
        ],
    )
    def pass_a(x_hbm, rowmax_hbm, gpart_hbm, xb, rmv, gv):
        wid = _wid()
        base = wid * rows_w
        lane = _lane_iota()

        def blk_body(b, g):
            pltpu.sync_copy(x_hbm.at[pl.ds((base + b * BLKR) * C, BLKR * C)],
                            xb)

            def row_body(r, carry):
                g, rmblk = carry

                def cb_body(cb, acc):
                    mx, mn = acc
                    o = r * C + cb * (U * L)
                    vs = [xb[pl.ds(o + u * L, L)] for u in range(U)]
                    return (jnp.maximum(mx, _tree(jnp.maximum, vs)),
                            jnp.minimum(mn, _tree(jnp.minimum, vs)))

                mxv, mnv = lax.fori_loop(
                    0, C // (U * L), cb_body,
                    (jnp.full((L,), -jnp.inf, jnp.float32),
                     jnp.full((L,), jnp.inf, jnp.float32)),
                )
                rmax = jnp.max(mxv)
                rmin = jnp.min(mnv)
                rmblk = jnp.where(lane == r, jnp.full((L,), rmax), rmblk)
                return (jnp.minimum(g, rmin - rmax), rmblk)

            g, rmblk = lax.fori_loop(
                0, BLKR, row_body, (g, jnp.zeros((L,), jnp.float32)))
            rmv[pl.ds(b * BLKR, L)] = rmblk
            return g

        g = lax.fori_loop(0, nblk, blk_body, jnp.float32(jnp.inf))
        gv[...] = jnp.full((L,), g)
        pltpu.sync_copy(rmv, rowmax_hbm.at[pl.ds(base, rows_w)])
        pltpu.sync_copy(gv, gpart_hbm.at[pl.ds(wid * L, L)])

    return pass_a


def _make_pass_b(R, C, NW, BLKR):
    rows_w = R // NW
    nblk = rows_w // BLKR

    @functools.partial(
        pl.kernel,
        out_type=(
            jax.ShapeDtypeStruct((R,), jnp.float32),
            jax.ShapeDtypeStruct((R,), jnp.float32),
        ),
        mesh=_mesh(),
        compiler_params=pltpu.CompilerParams(needs_layout_passes=False),
        scratch_types=[
            pltpu.VMEM((BLKR * C,), jnp.float32),
            pltpu.VMEM((rows_w,), jnp.float32),
            pltpu.VMEM((rows_w,), jnp.float32),
            pltpu.VMEM((rows_w,), jnp.float32),
            pltpu.VMEM((512 * L,), jnp.float32),
            pltpu.VMEM((2 * L,), jnp.float32),
        ],
    )
    def pass_b(x_hbm, rowmax_hbm, lut_hbm, scal_hbm, rowsum_hbm, emin_hbm,
               xb, rmv, rsv, emv, lutv, scv):
        wid = _wid()
        base = wid * rows_w
        lane = _lane_iota()
        pltpu.sync_copy(lut_hbm, lutv)
        pltpu.sync_copy(scal_hbm, scv)
        pltpu.sync_copy(rowmax_hbm.at[pl.ds(base, rows_w)], rmv)
        inv1 = scv[pl.ds(0, L)]
        zp1 = scv[pl.ds(L, L)]

        def blk_body(b, _):
            pltpu.sync_copy(x_hbm.at[pl.ds((base + b * BLKR) * C, BLKR * C)],
                            xb)

            def row_body(r, carry):
                rsblk, emblk = carry
                rm = _splat(rmv, b * BLKR + r)
                off = zp1 - rm * inv1

                def cb_body(cb, acc):
                    sacc, macc = acc
                    o = r * C + cb * (U * L)
                    es = []
                    for u in range(U):
                        v = xb[pl.ds(o + u * L, L)]
                        es.append(
                            plsc.load_gather(lutv,
                                             [_code16(v, inv1, off, lane)]))
                    return (sacc + _tree(jnp.add, es),
                            jnp.minimum(macc, _tree(jnp.minimum, es)))

                sacc, macc = lax.fori_loop(
                    0, C // (U * L), cb_body,
                    (jnp.zeros((L,), jnp.float32),
                     jnp.full((L,), jnp.inf, jnp.float32)),
                )
                rsum = jnp.sum(sacc)
                rmin = jnp.min(macc)
                rsblk = jnp.where(lane == r, jnp.full((L,), rsum), rsblk)
                emblk = jnp.where(lane == r, jnp.full((L,), rmin), emblk)
                return (rsblk, emblk)

            rsblk, emblk = lax.fori_loop(
                0, BLKR, row_body,
                (jnp.zeros((L,), jnp.float32), jnp.zeros((L,), jnp.float32)))
            rsv[pl.ds(b * BLKR, L)] = rsblk
            emv[pl.ds(b * BLKR, L)] = emblk
            return 0

        lax.fori_loop(0, nblk, blk_body, 0)
        pltpu.sync_copy(rsv, rowsum_hbm.at[pl.ds(base, rows_w)])
        pltpu.sync_copy(emv, emin_hbm.at[pl.ds(base, rows_w)])

    return pass_b


def _make_pass_c(R, C, NW, BLKR):
    rows_w = R // NW
    nblk = rows_w // BLKR

    @functools.partial(
        pl.kernel,
        out_type=jax.ShapeDtypeStruct((R * C,), jnp.float32),
        mesh=_mesh(),
        compiler_params=pltpu.CompilerParams(needs_layout_passes=False),
        scratch_types=[
            pltpu.VMEM((BLKR * C,), jnp.float32),
            pltpu.VMEM((BLKR * C,), jnp.float32),
            pltpu.VMEM((rows_w,), jnp.float32),
            pltpu.VMEM((rows_w,), jnp.float32),
            pltpu.VMEM((512 * L,), jnp.float32),
            pltpu.VMEM((5 * L,), jnp.float32),
        ],
    )
    def pass_c(x_hbm, rowmax_hbm, rowsum_hbm, lut_hbm, scal_hbm, out_hbm,
               xb, ob, rmv, rsv, lutv, scv):
        wid = _wid()
        base = wid * rows_w
        lane = _lane_iota()
        pltpu.sync_copy(lut_hbm, lutv)
        pltpu.sync_copy(scal_hbm, scv)
        pltpu.sync_copy(rowmax_hbm.at[pl.ds(base, rows_w)], rmv)
        pltpu.sync_copy(rowsum_hbm.at[pl.ds(base, rows_w)], rsv)
        inv1 = scv[pl.ds(0, L)]
        zp1 = scv[pl.ds(L, L)]
        inv2 = scv[pl.ds(2 * L, L)]
        zp2m = scv[pl.ds(3 * L, L)]
        s2 = scv[pl.ds(4 * L, L)]
        lo = np.float32(12582912.0 - 128.0)
        hi = np.float32(12582912.0 + 127.0)

        def blk_body(b, _):
            pltpu.sync_copy(x_hbm.at[pl.ds((base + b * BLKR) * C, BLKR * C)],
                            xb)

            def row_body(r, _):
                rm = _splat(rmv, b * BLKR + r)
                rs = _splat(rsv, b * BLKR + r)
                off = zp1 - rm * inv1
                k2 = (1.0 / rs) * inv2

                def cb_body(cb, _):
                    o = r * C + cb * (U * L)
                    for u in range(U):
                        v = xb[pl.ds(o + u * L, L)]
                        e = plsc.load_gather(
                            lutv, [_code16(v, inv1, off, lane)])
                        t2m = e * k2 + zp2m
                        q2m = jnp.clip(t2m, lo, hi)
                        ob[pl.ds(o + u * L, L)] = (q2m - zp2m) * s2
                    return 0

                lax.fori_loop(0, C // (U * L), cb_body, 0)
                return 0

            lax.fori_loop(0, BLKR, row_body, 0)
            pltpu.sync_copy(ob, out_hbm.at[pl.ds((base + b * BLKR) * C,
                                                 BLKR * C)])
            return 0

        lax.fori_loop(0, nblk, blk_body, 0)

    return pass_c


def kernel(inputs):
    B, H, S, C = inputs.shape
    R = B * H * S
    NW = 32
    x = inputs.reshape(R * C)

    rowmax, gpart = _make_pass_a(R, C, NW, L)(x)
    mn = jnp.min(gpart)
    mx = jnp.float32(0.0)
    scale1 = jnp.maximum((mx - mn) / 255.0, jnp.float32(1e-12))
    zp1 = jnp.clip(jnp.round(-128.0 - mn / scale1), -128, 127)
    codes = jnp.clip(jnp.arange(512, dtype=jnp.float32) - 256.0, 0.0, 255.0)
    lut = jnp.exp(((codes - 128.0) - zp1) * scale1)
    lut16 = jnp.repeat(lut, L)
    inv1 = 1.0 / scale1
    scal1 = jnp.concatenate(
        [jnp.full((L,), inv1), jnp.full((L,), zp1)]).astype(jnp.float32)

    rowsum, emin = _make_pass_b(R, C, NW, L)(x, rowmax, lut16, scal1)
    outmn = jnp.min(emin / rowsum)
    outmx = jnp.max(1.0 / rowsum)
    scale2 = jnp.maximum((outmx - outmn) / 255.0, jnp.float32(1e-12))
    zp2 = jnp.clip(jnp.round(-128.0 - outmn / scale2), -128, 127)
    scal2 = jnp.concatenate(
        [jnp.full((L,), inv1), jnp.full((L,), zp1),
         jnp.full((L,), 1.0 / scale2),
         jnp.full((L,), zp2 + jnp.float32(_MAGIC)),
         jnp.full((L,), scale2)]).astype(jnp.float32)

    out = _make_pass_c(R, C, NW, L)(x, rowmax, rowsum, lut16, scal2)
    return out.reshape(inputs.shape)


# speedup vs baseline: 5.8798x; 1.9812x over previous
import functools

import jax
import jax.numpy as jnp
import numpy as np
from jax import lax
from jax.experimental import pallas as pl
from jax.experimental.pallas import tpu as pltpu
from jax.experimental.pallas import tpu_sc as plsc

_MAGIC = np.float32(12582912.0)
_KMAGIC = 1262485504
_KME = _KMAGIC - 384
L = 16
U = 4


def _mesh():
    return plsc.VectorSubcoreMesh(core_axis_name="c", subcore_axis_name="s")


def _wid():
    info = plsc.get_sparse_core_info()
    return lax.axis_index("s") * info.num_cores + lax.axis_index("c")


def _tree(fn, vs):
    vs = list(vs)
    while len(vs) > 1:
        vs = [fn(vs[i], vs[i + 1]) for i in range(0, len(vs) - 1, 2)] + (
            [vs[-1]] if len(vs) % 2 else [])
    return vs[0]


def _lane_iota():
    return lax.iota(jnp.int32, L)


def _code16(v, inv1, off, lane):
    t = v * inv1 + off
    iu = lax.bitcast_convert_type(t + _MAGIC, jnp.int32)
    return jnp.left_shift(iu - _KME, 4) + lane


def _splat(ref, i):
    return plsc.load_gather(ref, [jnp.full((L,), 0, jnp.int32) + i])


def _make_pass_a(R, C, NW, BLKR):
    rows_w = R // NW
    nblk = rows_w // BLKR

    @functools.partial(
        pl.kernel,
        out_type=(
            jax.ShapeDtypeStruct((R,), jnp.float32),
            jax.ShapeDtypeStruct((NW * L,), jnp.float32),
        ),
        mesh=_mesh(),
        compiler_params=pltpu.CompilerParams(needs_layout_passes=False),
        scratch_types=[
            pltpu.VMEM((BLKR * C,), jnp.float32),
            pltpu.VMEM((rows_w,), jnp.float32),
            pltpu.VMEM((L,), jnp.float32),
        ],
    )
    def pass_a(x_hbm, rowmax_hbm, gpart_hbm, xb, rmv, gv):
        wid = _wid()
        base = wid * rows_w
        lane = _lane_iota()

        def blk_body(b, g):
            pltpu.sync_copy(x_hbm.at[pl.ds((base + b * BLKR) * C, BLKR * C)],
                            xb)

            def row_body(r, carry):
                g, rmblk = carry

                def cb_body(cb, acc):
                    mx, mn = acc
                    o = r * C + cb * (U * L)
                    vs = [xb[pl.ds(o + u * L, L)] for u in range(U)]
                    return (jnp.maximum(mx, _tree(jnp.maximum, vs)),
                            jnp.minimum(mn, _tree(jnp.minimum, vs)))

                mxv, mnv = plsc.parallel_loop(
                    0, C // (U * L), 1, unroll=2,
                    carry=(jnp.full((L,), -jnp.inf, jnp.float32),
                           jnp.full((L,), jnp.inf, jnp.float32)),
                )(cb_body)
                rmax = jnp.max(mxv)
                rmin = jnp.min(mnv)
                rmblk = jnp.where(lane == r, jnp.full((L,), rmax), rmblk)
                return (jnp.minimum(g, rmin - rmax), rmblk)

            g, rmblk = lax.fori_loop(
                0, BLKR, row_body, (g, jnp.zeros((L,), jnp.float32)))
            rmv[pl.ds(b * BLKR, L)] = rmblk
            return g

        g = lax.fori_loop(0, nblk, blk_body, jnp.float32(jnp.inf))
        gv[...] = jnp.full((L,), g)
        pltpu.sync_copy(rmv, rowmax_hbm.at[pl.ds(base, rows_w)])
        pltpu.sync_copy(gv, gpart_hbm.at[pl.ds(wid * L, L)])

    return pass_a


def _make_pass_b(R, C, NW, BLKR):
    rows_w = R // NW
    nblk = rows_w // BLKR

    @functools.partial(
        pl.kernel,
        out_type=(
            jax.ShapeDtypeStruct((R,), jnp.float32),
            jax.ShapeDtypeStruct((R,), jnp.float32),
        ),
        mesh=_mesh(),
        compiler_params=pltpu.CompilerParams(needs_layout_passes=False),
        scratch_types=[
            pltpu.VMEM((BLKR * C,), jnp.float32),
            pltpu.VMEM((rows_w,), jnp.float32),
            pltpu.VMEM((rows_w,), jnp.float32),
            pltpu.VMEM((rows_w,), jnp.float32),
            pltpu.VMEM((512 * L,), jnp.float32),
            pltpu.VMEM((2 * L,), jnp.float32),
        ],
    )
    def pass_b(x_hbm, rowmax_hbm, lut_hbm, scal_hbm, rowsum_hbm, emin_hbm,
               xb, rmv, rsv, emv, lutv, scv):
        wid = _wid()
        base = wid * rows_w
        lane = _lane_iota()
        pltpu.sync_copy(lut_hbm, lutv)
        pltpu.sync_copy(scal_hbm, scv)
        pltpu.sync_copy(rowmax_hbm.at[pl.ds(base, rows_w)], rmv)
        inv1 = scv[pl.ds(0, L)]
        zp1 = scv[pl.ds(L, L)]

        def blk_body(b, _):
            pltpu.sync_copy(x_hbm.at[pl.ds((base + b * BLKR) * C, BLKR * C)],
                            xb)

            def row_body(r, carry):
                rsblk, emblk = carry
                rm = _splat(rmv, b * BLKR + r)
                off = zp1 - rm * inv1

                def cb_body(cb, acc):
                    sacc, macc = acc
                    o = r * C + cb * (U * L)
                    es = []
                    for u in range(U):
                        v = xb[pl.ds(o + u * L, L)]
                        es.append(
                            plsc.load_gather(lutv,
                                             [_code16(v, inv1, off, lane)]))
                    return (sacc + _tree(jnp.add, es),
                            jnp.minimum(macc, _tree(jnp.minimum, es)))

                sacc, macc = plsc.parallel_loop(
                    0, C // (U * L), 1, unroll=2,
                    carry=(jnp.zeros((L,), jnp.float32),
                           jnp.full((L,), jnp.inf, jnp.float32)),
                )(cb_body)
                rsum = jnp.sum(sacc)
                rmin = jnp.min(macc)
                rsblk = jnp.where(lane == r, jnp.full((L,), rsum), rsblk)
                emblk = jnp.where(lane == r, jnp.full((L,), rmin), emblk)
                return (rsblk, emblk)

            rsblk, emblk = lax.fori_loop(
                0, BLKR, row_body,
                (jnp.zeros((L,), jnp.float32), jnp.zeros((L,), jnp.float32)))
            rsv[pl.ds(b * BLKR, L)] = rsblk
            emv[pl.ds(b * BLKR, L)] = emblk
            return 0

        lax.fori_loop(0, nblk, blk_body, 0)
        pltpu.sync_copy(rsv, rowsum_hbm.at[pl.ds(base, rows_w)])
        pltpu.sync_copy(emv, emin_hbm.at[pl.ds(base, rows_w)])

    return pass_b


def _make_pass_c(R, C, NW, BLKR):
    rows_w = R // NW
    nblk = rows_w // BLKR

    @functools.partial(
        pl.kernel,
        out_type=jax.ShapeDtypeStruct((R * C,), jnp.float32),
        mesh=_mesh(),
        compiler_params=pltpu.CompilerParams(needs_layout_passes=False),
        scratch_types=[
            pltpu.VMEM((BLKR * C,), jnp.float32),
            pltpu.VMEM((BLKR * C,), jnp.float32),
            pltpu.VMEM((rows_w,), jnp.float32),
            pltpu.VMEM((rows_w,), jnp.float32),
            pltpu.VMEM((512 * L,), jnp.float32),
            pltpu.VMEM((5 * L,), jnp.float32),
        ],
    )
    def pass_c(x_hbm, rowmax_hbm, rowsum_hbm, lut_hbm, scal_hbm, out_hbm,
               xb, ob, rmv, rsv, lutv, scv):
        wid = _wid()
        base = wid * rows_w
        lane = _lane_iota()
        pltpu.sync_copy(lut_hbm, lutv)
        pltpu.sync_copy(scal_hbm, scv)
        pltpu.sync_copy(rowmax_hbm.at[pl.ds(base, rows_w)], rmv)
        pltpu.sync_copy(rowsum_hbm.at[pl.ds(base, rows_w)], rsv)
        inv1 = scv[pl.ds(0, L)]
        zp1 = scv[pl.ds(L, L)]
        inv2 = scv[pl.ds(2 * L, L)]
        zp2m = scv[pl.ds(3 * L, L)]
        s2 = scv[pl.ds(4 * L, L)]
        lo = np.float32(12582912.0 - 128.0)
        hi = np.float32(12582912.0 + 127.0)

        def blk_body(b, _):
            pltpu.sync_copy(x_hbm.at[pl.ds((base + b * BLKR) * C, BLKR * C)],
                            xb)

            def row_body(r, _):
                rm = _splat(rmv, b * BLKR + r)
                rs = _splat(rsv, b * BLKR + r)
                off = zp1 - rm * inv1
                k2 = (1.0 / rs) * inv2

                def cb_body(cb):
                    o = r * C + cb * (U * L)
                    for u in range(U):
                        v = xb[pl.ds(o + u * L, L)]
                        e = plsc.load_gather(
                            lutv, [_code16(v, inv1, off, lane)])
                        t2m = e * k2 + zp2m
                        q2m = jnp.clip(t2m, lo, hi)
                        ob[pl.ds(o + u * L, L)] = (q2m - zp2m) * s2

                plsc.parallel_loop(0, C // (U * L), 1, unroll=2)(cb_body)
                return 0

            lax.fori_loop(0, BLKR, row_body, 0)
            pltpu.sync_copy(ob, out_hbm.at[pl.ds((base + b * BLKR) * C,
                                                 BLKR * C)])
            return 0

        lax.fori_loop(0, nblk, blk_body, 0)

    return pass_c


def kernel(inputs):
    B, H, S, C = inputs.shape
    R = B * H * S
    NW = 32
    x = inputs.reshape(R * C)

    rowmax, gpart = _make_pass_a(R, C, NW, L)(x)
    mn = jnp.min(gpart)
    mx = jnp.float32(0.0)
    scale1 = jnp.maximum((mx - mn) / 255.0, jnp.float32(1e-12))
    zp1 = jnp.clip(jnp.round(-128.0 - mn / scale1), -128, 127)
    codes = jnp.clip(jnp.arange(512, dtype=jnp.float32) - 256.0, 0.0, 255.0)
    lut = jnp.exp(((codes - 128.0) - zp1) * scale1)
    lut16 = jnp.repeat(lut, L)
    inv1 = 1.0 / scale1
    scal1 = jnp.concatenate(
        [jnp.full((L,), inv1), jnp.full((L,), zp1)]).astype(jnp.float32)

    rowsum, emin = _make_pass_b(R, C, NW, L)(x, rowmax, lut16, scal1)
    outmn = jnp.min(emin / rowsum)
    outmx = jnp.max(1.0 / rowsum)
    scale2 = jnp.maximum((outmx - outmn) / 255.0, jnp.float32(1e-12))
    zp2 = jnp.clip(jnp.round(-128.0 - outmn / scale2), -128, 127)
    scal2 = jnp.concatenate(
        [jnp.full((L,), inv1), jnp.full((L,), zp1),
         jnp.full((L,), 1.0 / scale2),
         jnp.full((L,), zp2 + jnp.float32(_MAGIC)),
         jnp.full((L,), scale2)]).astype(jnp.float32)

    out = _make_pass_c(R, C, NW, L)(x, rowmax, rowsum, lut16, scal2)
    return out.reshape(inputs.shape)


# speedup vs baseline: 13.0026x; 2.2114x over previous
import functools

import jax
import jax.numpy as jnp
import numpy as np
from jax import lax
from jax.experimental import pallas as pl
from jax.experimental.pallas import tpu as pltpu
from jax.experimental.pallas import tpu_sc as plsc

_MAGIC = np.float32(12582912.0)
_KMAGIC = 1262485504
_KME = _KMAGIC - 384
L = 16
U = 4


def _mesh():
    return plsc.VectorSubcoreMesh(core_axis_name="c", subcore_axis_name="s")


def _wid():
    info = plsc.get_sparse_core_info()
    return lax.axis_index("s") * info.num_cores + lax.axis_index("c")


def _tree(fn, vs):
    vs = list(vs)
    while len(vs) > 1:
        vs = [fn(vs[i], vs[i + 1]) for i in range(0, len(vs) - 1, 2)] + (
            [vs[-1]] if len(vs) % 2 else [])
    return vs[0]


def _lane_iota():
    return lax.iota(jnp.int32, L)


def _code16(v, inv1, off, lane):
    t = v * inv1 + off
    iu = lax.bitcast_convert_type(t + _MAGIC, jnp.int32)
    return jnp.left_shift(iu - _KME, 4) + lane


def _splat(ref, i):
    return plsc.load_gather(ref, [jnp.full((L,), 0, jnp.int32) + i])


def _hs(row, sh, mask):
    s = jnp.bitwise_and(row, mask)
    return jnp.right_shift(row, sh), pl.multiple_of(s, 8)


def _make_pass_a(shape, NW, BLKR):
    _, H, S, C = shape
    R = H * S
    rows_w = R // NW
    nblk = rows_w // BLKR
    sh, mask = (S - 1).bit_length(), S - 1

    @functools.partial(
        pl.kernel,
        out_type=(
            jax.ShapeDtypeStruct((R,), jnp.float32),
            jax.ShapeDtypeStruct((NW * L,), jnp.float32),
        ),
        mesh=_mesh(),
        compiler_params=pltpu.CompilerParams(needs_layout_passes=False),
        scratch_types=[
            pltpu.VMEM((BLKR, C), jnp.float32),
            pltpu.VMEM((BLKR, C), jnp.float32),
            pltpu.VMEM((rows_w,), jnp.float32),
            pltpu.VMEM((L,), jnp.float32),
            pltpu.SemaphoreType.DMA,
            pltpu.SemaphoreType.DMA,
        ],
    )
    def pass_a(x4, rowmax_hbm, gpart_hbm, xb0, xb1, rmv, gv, s0, s1):
        wid = _wid()
        base = wid * rows_w
        lane = _lane_iota()

        def fill(b, buf, sem):
            h, s = _hs(base + b * BLKR, sh, mask)
            pltpu.async_copy(x4.at[0, h, pl.ds(s, BLKR), :], buf, sem)

        def wait_fill(buf, sem):
            pltpu.make_async_copy(
                x4.at[0, 0, pl.ds(0, BLKR), :], buf, sem).wait()

        def process(b, buf, g):
            def row_body(r, carry):
                g, rmblk = carry

                def cb_body(cb, acc):
                    mx, mn = acc
                    o = cb * (U * L)
                    vs = [buf[r, pl.ds(o + u * L, L)] for u in range(U)]
                    return (jnp.maximum(mx, _tree(jnp.maximum, vs)),
                            jnp.minimum(mn, _tree(jnp.minimum, vs)))

                mxv, mnv = plsc.parallel_loop(
                    0, C // (U * L), 1, unroll=2,
                    carry=(jnp.full((L,), -jnp.inf, jnp.float32),
                           jnp.full((L,), jnp.inf, jnp.float32)),
                )(cb_body)
                rmax = jnp.max(mxv)
                rmin = jnp.min(mnv)
                rmblk = jnp.where(lane == r, jnp.full((L,), rmax), rmblk)
                return (jnp.minimum(g, rmin - rmax), rmblk)

            g, rmblk = lax.fori_loop(
                0, BLKR, row_body, (g, jnp.zeros((L,), jnp.float32)))
            rmv[pl.ds(b * BLKR, L)] = rmblk
            return g

        fill(0, xb0, s0)
        fill(1, xb1, s1)

        def outer(gi, g):
            b0 = gi * 2

            wait_fill(xb0, s0)
            g = process(b0, xb0, g)

            @pl.when(b0 + 2 < nblk)
            def _():
                fill(b0 + 2, xb0, s0)

            wait_fill(xb1, s1)
            g = process(b0 + 1, xb1, g)

            @pl.when(b0 + 3 < nblk)
            def _():
                fill(b0 + 3, xb1, s1)

            return g

        g = lax.fori_loop(0, nblk // 2, outer, jnp.float32(jnp.inf))
        gv[...] = jnp.full((L,), g)
        pltpu.sync_copy(rmv, rowmax_hbm.at[pl.ds(base, rows_w)])
        pltpu.sync_copy(gv, gpart_hbm.at[pl.ds(wid * L, L)])

    return pass_a


def _make_pass_b(shape, NW, BLKR):
    _, H, S, C = shape
    R = H * S
    rows_w = R // NW
    nblk = rows_w // BLKR
    sh, mask = (S - 1).bit_length(), S - 1

    @functools.partial(
        pl.kernel,
        out_type=(
            jax.ShapeDtypeStruct((R,), jnp.float32),
            jax.ShapeDtypeStruct((R,), jnp.float32),
        ),
        mesh=_mesh(),
        compiler_params=pltpu.CompilerParams(needs_layout_passes=False),
        scratch_types=[
            pltpu.VMEM((BLKR, C), jnp.float32),
            pltpu.VMEM((BLKR, C), jnp.float32),
            pltpu.VMEM((rows_w,), jnp.float32),
            pltpu.VMEM((rows_w,), jnp.float32),
            pltpu.VMEM((rows_w,), jnp.float32),
            pltpu.VMEM((512 * L,), jnp.float32),
            pltpu.VMEM((2 * L,), jnp.float32),
            pltpu.SemaphoreType.DMA,
            pltpu.SemaphoreType.DMA,
        ],
    )
    def pass_b(x4, rowmax_hbm, lut_hbm, scal_hbm, rowsum_hbm, emin_hbm,
               xb0, xb1, rmv, rsv, emv, lutv, scv, s0, s1):
        wid = _wid()
        base = wid * rows_w
        lane = _lane_iota()
        pltpu.sync_copy(lut_hbm, lutv)
        pltpu.sync_copy(scal_hbm, scv)
        pltpu.sync_copy(rowmax_hbm.at[pl.ds(base, rows_w)], rmv)
        inv1 = scv[pl.ds(0, L)]
        zp1 = scv[pl.ds(L, L)]

        def fill(b, buf, sem):
            h, s = _hs(base + b * BLKR, sh, mask)
            pltpu.async_copy(x4.at[0, h, pl.ds(s, BLKR), :], buf, sem)

        def wait_fill(buf, sem):
            pltpu.make_async_copy(
                x4.at[0, 0, pl.ds(0, BLKR), :], buf, sem).wait()

        def process(b, buf):
            def row_body(r, carry):
                rsblk, emblk = carry
                rm = _splat(rmv, b * BLKR + r)
                off = zp1 - rm * inv1

                def cb_body(cb, acc):
                    sacc, macc = acc
                    o = cb * (U * L)
                    es = []
                    for u in range(U):
                        v = buf[r, pl.ds(o + u * L, L)]
                        es.append(
                            plsc.load_gather(lutv,
                                             [_code16(v, inv1, off, lane)]))
                    return (sacc + _tree(jnp.add, es),
                            jnp.minimum(macc, _tree(jnp.minimum, es)))

                sacc, macc = plsc.parallel_loop(
                    0, C // (U * L), 1, unroll=2,
                    carry=(jnp.zeros((L,), jnp.float32),
                           jnp.full((L,), jnp.inf, jnp.float32)),
                )(cb_body)
                rsum = jnp.sum(sacc)
                rmin = jnp.min(macc)
                rsblk = jnp.where(lane == r, jnp.full((L,), rsum), rsblk)
                emblk = jnp.where(lane == r, jnp.full((L,), rmin), emblk)
                return (rsblk, emblk)

            rsblk, emblk = lax.fori_loop(
                0, BLKR, row_body,
                (jnp.zeros((L,), jnp.float32), jnp.zeros((L,), jnp.float32)))
            rsv[pl.ds(b * BLKR, L)] = rsblk
            emv[pl.ds(b * BLKR, L)] = emblk

        fill(0, xb0, s0)
        fill(1, xb1, s1)

        def outer(gi, _):
            b0 = gi * 2

            wait_fill(xb0, s0)
            process(b0, xb0)

            @pl.when(b0 + 2 < nblk)
            def _():
                fill(b0 + 2, xb0, s0)

            wait_fill(xb1, s1)
            process(b0 + 1, xb1)

            @pl.when(b0 + 3 < nblk)
            def _():
                fill(b0 + 3, xb1, s1)

            return 0

        lax.fori_loop(0, nblk // 2, outer, 0)
        pltpu.sync_copy(rsv, rowsum_hbm.at[pl.ds(base, rows_w)])
        pltpu.sync_copy(emv, emin_hbm.at[pl.ds(base, rows_w)])

    return pass_b


def _make_pass_c(shape, NW, BLKR):
    _, H, S, C = shape
    R = H * S
    rows_w = R // NW
    nblk = rows_w // BLKR
    sh, mask = (S - 1).bit_length(), S - 1

    @functools.partial(
        pl.kernel,
        out_type=jax.ShapeDtypeStruct(shape, jnp.float32),
        mesh=_mesh(),
        compiler_params=pltpu.CompilerParams(needs_layout_passes=False),
        scratch_types=[
            pltpu.VMEM((BLKR, C), jnp.float32),
            pltpu.VMEM((BLKR, C), jnp.float32),
            pltpu.VMEM((BLKR, C), jnp.float32),
            pltpu.VMEM((BLKR, C), jnp.float32),
            pltpu.VMEM((rows_w,), jnp.float32),
            pltpu.VMEM((rows_w,), jnp.float32),
            pltpu.VMEM((512 * L,), jnp.float32),
            pltpu.VMEM((5 * L,), jnp.float32),
            pltpu.SemaphoreType.DMA,
            pltpu.SemaphoreType.DMA,
            pltpu.SemaphoreType.DMA,
            pltpu.SemaphoreType.DMA,
        ],
    )
    def pass_c(x4, rowmax_hbm, rowsum_hbm, lut_hbm, scal_hbm, out4,
               xb0, xb1, ob0, ob1, rmv, rsv, lutv, scv, si0, si1, so0, so1):
        wid = _wid()
        base = wid * rows_w
        lane = _lane_iota()
        pltpu.sync_copy(lut_hbm, lutv)
        pltpu.sync_copy(scal_hbm, scv)
        pltpu.sync_copy(rowmax_hbm.at[pl.ds(base, rows_w)], rmv)
        pltpu.sync_copy(rowsum_hbm.at[pl.ds(base, rows_w)], rsv)
        inv1 = scv[pl.ds(0, L)]
        zp1 = scv[pl.ds(L, L)]
        inv2 = scv[pl.ds(2 * L, L)]
        zp2m = scv[pl.ds(3 * L, L)]
        s2 = scv[pl.ds(4 * L, L)]
        lo = np.float32(12582912.0 - 128.0)
        hi = np.float32(12582912.0 + 127.0)

        def fill(b, buf, sem):
            h, s = _hs(base + b * BLKR, sh, mask)
            pltpu.async_copy(x4.at[0, h, pl.ds(s, BLKR), :], buf, sem)

        def wait_fill(buf, sem):
            pltpu.make_async_copy(
                x4.at[0, 0, pl.ds(0, BLKR), :], buf, sem).wait()

        def drain(b, buf, sem):
            h, s = _hs(base + b * BLKR, sh, mask)
            pltpu.async_copy(buf, out4.at[0, h, pl.ds(s, BLKR), :], sem)

        def wait_drain(buf, sem):
            pltpu.make_async_copy(
                buf, out4.at[0, 0, pl.ds(0, BLKR), :], sem).wait()

        def process(b, buf, obuf):
            def row_body(r, _):
                rm = _splat(rmv, b * BLKR + r)
                rs = _splat(rsv, b * BLKR + r)
                off = zp1 - rm * inv1
                k2 = (1.0 / rs) * inv2

                def cb_body(cb):
                    o = cb * (U * L)
                    for u in range(U):
                        v = buf[r, pl.ds(o + u * L, L)]
                        e = plsc.load_gather(
                            lutv, [_code16(v, inv1, off, lane)])
                        t2m = e * k2 + zp2m
                        q2m = jnp.clip(t2m, lo, hi)
                        obuf[r, pl.ds(o + u * L, L)] = (q2m - zp2m) * s2

                plsc.parallel_loop(0, C // (U * L), 1, unroll=2)(cb_body)
                return 0

            lax.fori_loop(0, BLKR, row_body, 0)

        fill(0, xb0, si0)
        fill(1, xb1, si1)

        def outer(gi, _):
            b0 = gi * 2

            wait_fill(xb0, si0)

            @pl.when(b0 >= 2)
            def _():
                wait_drain(ob0, so0)

            process(b0, xb0, ob0)
            drain(b0, ob0, so0)

            @pl.when(b0 + 2 < nblk)
            def _():
                fill(b0 + 2, xb0, si0)

            wait_fill(xb1, si1)

            @pl.when(b0 >= 2)
            def _():
                wait_drain(ob1, so1)

            process(b0 + 1, xb1, ob1)
            drain(b0 + 1, ob1, so1)

            @pl.when(b0 + 3 < nblk)
            def _():
                fill(b0 + 3, xb1, si1)

            return 0

        lax.fori_loop(0, nblk // 2, outer, 0)
        wait_drain(ob0, so0)
        wait_drain(ob1, so1)

    return pass_c


def kernel(inputs):
    shape = inputs.shape
    NW = 32

    rowmax, gpart = _make_pass_a(shape, NW, L)(inputs)
    mn = jnp.min(gpart)
    mx = jnp.float32(0.0)
    scale1 = jnp.maximum((mx - mn) / 255.0, jnp.float32(1e-12))
    zp1 = jnp.clip(jnp.round(-128.0 - mn / scale1), -128, 127)
    codes = jnp.clip(jnp.arange(512, dtype=jnp.float32) - 256.0, 0.0, 255.0)
    lut = jnp.exp(((codes - 128.0) - zp1) * scale1)
    lut16 = jnp.repeat(lut, L)
    inv1 = 1.0 / scale1
    scal1 = jnp.concatenate(
        [jnp.full((L,), inv1), jnp.full((L,), zp1)]).astype(jnp.float32)

    rowsum, emin = _make_pass_b(shape, NW, L)(inputs, rowmax, lut16, scal1)
    outmn = jnp.min(emin / rowsum)
    outmx = jnp.max(1.0 / rowsum)
    scale2 = jnp.maximum((outmx - outmn) / 255.0, jnp.float32(1e-12))
    zp2 = jnp.clip(jnp.round(-128.0 - outmn / scale2), -128, 127)
    scal2 = jnp.concatenate(
        [jnp.full((L,), inv1), jnp.full((L,), zp1),
         jnp.full((L,), 1.0 / scale2),
         jnp.full((L,), zp2 + jnp.float32(_MAGIC)),
         jnp.full((L,), scale2)]).astype(jnp.float32)

    return _make_pass_c(shape, NW, L // 2)(inputs, rowmax, rowsum, lut16,
                                           scal2)


# speedup vs baseline: 13.4888x; 1.0374x over previous
import functools

import jax
import jax.numpy as jnp
import numpy as np
from jax import lax
from jax.experimental import pallas as pl
from jax.experimental.pallas import tpu as pltpu
from jax.experimental.pallas import tpu_sc as plsc

_MAGIC = np.float32(12582912.0)
_KMAGIC = 1262485504
_KME = _KMAGIC - 384
L = 16
U = 4


def _mesh():
    return plsc.VectorSubcoreMesh(core_axis_name="c", subcore_axis_name="s")


def _wid():
    info = plsc.get_sparse_core_info()
    return lax.axis_index("s") * info.num_cores + lax.axis_index("c")


def _tree(fn, vs):
    vs = list(vs)
    while len(vs) > 1:
        vs = [fn(vs[i], vs[i + 1]) for i in range(0, len(vs) - 1, 2)] + (
            [vs[-1]] if len(vs) % 2 else [])
    return vs[0]


def _lane_iota():
    return lax.iota(jnp.int32, L)


def _code16(v, inv1, off, lane):
    t = v * inv1 + off
    iu = lax.bitcast_convert_type(t + _MAGIC, jnp.int32)
    return jnp.left_shift(iu - _KME, 4) + lane


def _splat(ref, i):
    return plsc.load_gather(ref, [jnp.full((L,), 0, jnp.int32) + i])


def _hs(row, sh, mask):
    s = jnp.bitwise_and(row, mask)
    return jnp.right_shift(row, sh), pl.multiple_of(s, 8)


def _make_pass_a(shape, NW, BLKR):
    _, H, S, C = shape
    R = H * S
    rows_w = R // NW
    nblk = rows_w // BLKR
    sh, mask = (S - 1).bit_length(), S - 1

    @functools.partial(
        pl.kernel,
        out_type=(
            jax.ShapeDtypeStruct((R,), jnp.float32),
            jax.ShapeDtypeStruct((NW * L,), jnp.float32),
        ),
        mesh=_mesh(),
        compiler_params=pltpu.CompilerParams(needs_layout_passes=False),
        scratch_types=[
            pltpu.VMEM((BLKR, C), jnp.float32),
            pltpu.VMEM((BLKR, C), jnp.float32),
            pltpu.VMEM((rows_w,), jnp.float32),
            pltpu.VMEM((L,), jnp.float32),
            pltpu.SemaphoreType.DMA,
            pltpu.SemaphoreType.DMA,
        ],
    )
    def pass_a(x4, rowmax_hbm, gpart_hbm, xb0, xb1, rmv, gv, s0, s1):
        wid = _wid()
        base = wid * rows_w
        lane = _lane_iota()

        def fill(b, buf, sem):
            h, s = _hs(base + b * BLKR, sh, mask)
            pltpu.async_copy(x4.at[0, h, pl.ds(s, BLKR), :], buf, sem)

        def wait_fill(buf, sem):
            pltpu.make_async_copy(
                x4.at[0, 0, pl.ds(0, BLKR), :], buf, sem).wait()

        def process(b, buf, g):
            def row_body(r, carry):
                g, rmblk = carry

                def cb_body(cb, acc):
                    mx, mn = acc
                    o = cb * (U * L)
                    vs = [buf[r, pl.ds(o + u * L, L)] for u in range(U)]
                    return (jnp.maximum(mx, _tree(jnp.maximum, vs)),
                            jnp.minimum(mn, _tree(jnp.minimum, vs)))

                mxv, mnv = plsc.parallel_loop(
                    0, C // (U * L), 1, unroll=2,
                    carry=(jnp.full((L,), -jnp.inf, jnp.float32),
                           jnp.full((L,), jnp.inf, jnp.float32)),
                )(cb_body)
                rmax = jnp.max(mxv)
                rmin = jnp.min(mnv)
                rmblk = jnp.where(lane == r, jnp.full((L,), rmax), rmblk)
                return (jnp.minimum(g, rmin - rmax), rmblk)

            g, rmblk = lax.fori_loop(
                0, BLKR, row_body, (g, jnp.zeros((L,), jnp.float32)))
            rmv[pl.ds(b * BLKR, L)] = rmblk
            return g

        fill(0, xb0, s0)
        fill(1, xb1, s1)

        def outer(gi, g):
            b0 = gi * 2

            wait_fill(xb0, s0)
            g = process(b0, xb0, g)

            @pl.when(b0 + 2 < nblk)
            def _():
                fill(b0 + 2, xb0, s0)

            wait_fill(xb1, s1)
            g = process(b0 + 1, xb1, g)

            @pl.when(b0 + 3 < nblk)
            def _():
                fill(b0 + 3, xb1, s1)

            return g

        g = lax.fori_loop(0, nblk // 2, outer, jnp.float32(jnp.inf))
        gv[...] = jnp.full((L,), g)
        pltpu.sync_copy(rmv, rowmax_hbm.at[pl.ds(base, rows_w)])
        pltpu.sync_copy(gv, gpart_hbm.at[pl.ds(wid * L, L)])

    return pass_a


def _make_pass_b(shape, NW, BLKR):
    _, H, S, C = shape
    R = H * S
    rows_w = R // NW
    nblk = rows_w // BLKR
    sh, mask = (S - 1).bit_length(), S - 1

    @functools.partial(
        pl.kernel,
        out_type=(
            jax.ShapeDtypeStruct((R,), jnp.float32),
            jax.ShapeDtypeStruct((R,), jnp.float32),
        ),
        mesh=_mesh(),
        compiler_params=pltpu.CompilerParams(needs_layout_passes=False),
        scratch_types=[
            pltpu.VMEM((BLKR, C), jnp.float32),
            pltpu.VMEM((BLKR, C), jnp.float32),
            pltpu.VMEM((rows_w,), jnp.float32),
            pltpu.VMEM((rows_w,), jnp.float32),
            pltpu.VMEM((rows_w,), jnp.float32),
            pltpu.VMEM((512 * L,), jnp.float32),
            pltpu.VMEM((2 * L,), jnp.float32),
            pltpu.SemaphoreType.DMA,
            pltpu.SemaphoreType.DMA,
        ],
    )
    def pass_b(x4, rowmax_hbm, lut_hbm, scal_hbm, rowsum_hbm, emin_hbm,
               xb0, xb1, rmv, rsv, emv, lutv, scv, s0, s1):
        wid = _wid()
        base = wid * rows_w
        lane = _lane_iota()
        pltpu.sync_copy(lut_hbm, lutv)
        pltpu.sync_copy(scal_hbm, scv)
        pltpu.sync_copy(rowmax_hbm.at[pl.ds(base, rows_w)], rmv)
        inv1 = scv[pl.ds(0, L)]
        zp1 = scv[pl.ds(L, L)]

        def fill(b, buf, sem):
            h, s = _hs(base + b * BLKR, sh, mask)
            pltpu.async_copy(x4.at[0, h, pl.ds(s, BLKR), :], buf, sem)

        def wait_fill(buf, sem):
            pltpu.make_async_copy(
                x4.at[0, 0, pl.ds(0, BLKR), :], buf, sem).wait()

        def process(b, buf):
            def row_body(r, carry):
                rsblk, emblk = carry
                rm = _splat(rmv, b * BLKR + r)
                off = zp1 - rm * inv1

                def cb_body(cb, acc):
                    sacc, macc = acc
                    o = cb * (U * L)
                    es = []
                    for u in range(U):
                        v = buf[r, pl.ds(o + u * L, L)]
                        es.append(
                            plsc.load_gather(lutv,
                                             [_code16(v, inv1, off, lane)]))
                    return (sacc + _tree(jnp.add, es),
                            jnp.minimum(macc, _tree(jnp.minimum, es)))

                sacc, macc = plsc.parallel_loop(
                    0, C // (U * L), 1, unroll=4,
                    carry=(jnp.zeros((L,), jnp.float32),
                           jnp.full((L,), jnp.inf, jnp.float32)),
                )(cb_body)
                rsum = jnp.sum(sacc)
                rmin = jnp.min(macc)
                rsblk = jnp.where(lane == r, jnp.full((L,), rsum), rsblk)
                emblk = jnp.where(lane == r, jnp.full((L,), rmin), emblk)
                return (rsblk, emblk)

            rsblk, emblk = lax.fori_loop(
                0, BLKR, row_body,
                (jnp.zeros((L,), jnp.float32), jnp.zeros((L,), jnp.float32)))
            rsv[pl.ds(b * BLKR, L)] = rsblk
            emv[pl.ds(b * BLKR, L)] = emblk

        fill(0, xb0, s0)
        fill(1, xb1, s1)

        def outer(gi, _):
            b0 = gi * 2

            wait_fill(xb0, s0)
            process(b0, xb0)

            @pl.when(b0 + 2 < nblk)
            def _():
                fill(b0 + 2, xb0, s0)

            wait_fill(xb1, s1)
            process(b0 + 1, xb1)

            @pl.when(b0 + 3 < nblk)
            def _():
                fill(b0 + 3, xb1, s1)

            return 0

        lax.fori_loop(0, nblk // 2, outer, 0)
        pltpu.sync_copy(rsv, rowsum_hbm.at[pl.ds(base, rows_w)])
        pltpu.sync_copy(emv, emin_hbm.at[pl.ds(base, rows_w)])

    return pass_b


def _make_pass_c(shape, NW, BLKR):
    _, H, S, C = shape
    R = H * S
    rows_w = R // NW
    nblk = rows_w // BLKR
    sh, mask = (S - 1).bit_length(), S - 1

    @functools.partial(
        pl.kernel,
        out_type=jax.ShapeDtypeStruct(shape, jnp.float32),
        mesh=_mesh(),
        compiler_params=pltpu.CompilerParams(needs_layout_passes=False),
        scratch_types=[
            pltpu.VMEM((BLKR, C), jnp.float32),
            pltpu.VMEM((BLKR, C), jnp.float32),
            pltpu.VMEM((BLKR, C), jnp.float32),
            pltpu.VMEM((BLKR, C), jnp.float32),
            pltpu.VMEM((rows_w,), jnp.float32),
            pltpu.VMEM((rows_w,), jnp.float32),
            pltpu.VMEM((512 * L,), jnp.float32),
            pltpu.VMEM((5 * L,), jnp.float32),
            pltpu.SemaphoreType.DMA,
            pltpu.SemaphoreType.DMA,
            pltpu.SemaphoreType.DMA,
            pltpu.SemaphoreType.DMA,
        ],
    )
    def pass_c(x4, rowmax_hbm, invsum_hbm, lut_hbm, scal_hbm, out4,
               xb0, xb1, ob0, ob1, rmv, rsv, lutv, scv, si0, si1, so0, so1):
        wid = _wid()
        base = wid * rows_w
        lane = _lane_iota()
        pltpu.sync_copy(lut_hbm, lutv)
        pltpu.sync_copy(scal_hbm, scv)
        pltpu.sync_copy(rowmax_hbm.at[pl.ds(base, rows_w)], rmv)
        pltpu.sync_copy(invsum_hbm.at[pl.ds(base, rows_w)], rsv)
        inv1 = scv[pl.ds(0, L)]
        zp1 = scv[pl.ds(L, L)]
        inv2 = scv[pl.ds(2 * L, L)]
        zp2m = scv[pl.ds(3 * L, L)]
        s2 = scv[pl.ds(4 * L, L)]
        hi = np.float32(12582912.0 + 127.0)

        def fill(b, buf, sem):
            h, s = _hs(base + b * BLKR, sh, mask)
            pltpu.async_copy(x4.at[0, h, pl.ds(s, BLKR), :], buf, sem)

        def wait_fill(buf, sem):
            pltpu.make_async_copy(
                x4.at[0, 0, pl.ds(0, BLKR), :], buf, sem).wait()

        def drain(b, buf, sem):
            h, s = _hs(base + b * BLKR, sh, mask)
            pltpu.async_copy(buf, out4.at[0, h, pl.ds(s, BLKR), :], sem)

        def wait_drain(buf, sem):
            pltpu.make_async_copy(
                buf, out4.at[0, 0, pl.ds(0, BLKR), :], sem).wait()

        def process(b, buf, obuf):
            def row_body(r, _):
                rm = _splat(rmv, b * BLKR + r)
                rs = _splat(rsv, b * BLKR + r)
                off = zp1 - rm * inv1
                k2 = rs * inv2

                def cb_body(cb):
                    o = cb * (U * L)
                    for u in range(U):
                        v = buf[r, pl.ds(o + u * L, L)]
                        e = plsc.load_gather(
                            lutv, [_code16(v, inv1, off, lane)])
                        t2m = e * k2 + zp2m
                        q2m = jnp.minimum(t2m, hi)
                        obuf[r, pl.ds(o + u * L, L)] = (q2m - zp2m) * s2

                plsc.parallel_loop(0, C // (U * L), 1, unroll=4)(cb_body)
                return 0

            lax.fori_loop(0, BLKR, row_body, 0)

        fill(0, xb0, si0)
        fill(1, xb1, si1)

        def outer(gi, _):
            b0 = gi * 2

            wait_fill(xb0, si0)

            @pl.when(b0 >= 2)
            def _():
                wait_drain(ob0, so0)

            process(b0, xb0, ob0)
            drain(b0, ob0, so0)

            @pl.when(b0 + 2 < nblk)
            def _():
                fill(b0 + 2, xb0, si0)

            wait_fill(xb1, si1)

            @pl.when(b0 >= 2)
            def _():
                wait_drain(ob1, so1)

            process(b0 + 1, xb1, ob1)
            drain(b0 + 1, ob1, so1)

            @pl.when(b0 + 3 < nblk)
            def _():
                fill(b0 + 3, xb1, si1)

            return 0

        lax.fori_loop(0, nblk // 2, outer, 0)
        wait_drain(ob0, so0)
        wait_drain(ob1, so1)

    return pass_c


def kernel(inputs):
    shape = inputs.shape
    NW = 32

    rowmax, gpart = _make_pass_a(shape, NW, L)(inputs)
    mn = jnp.min(gpart)
    mx = jnp.float32(0.0)
    scale1 = jnp.maximum((mx - mn) / 255.0, jnp.float32(1e-12))
    zp1 = jnp.clip(jnp.round(-128.0 - mn / scale1), -128, 127)
    codes = jnp.clip(jnp.arange(512, dtype=jnp.float32) - 256.0, 0.0, 255.0)
    lut = jnp.exp(((codes - 128.0) - zp1) * scale1)
    lut16 = jnp.repeat(lut, L)
    inv1 = 1.0 / scale1
    scal1 = jnp.concatenate(
        [jnp.full((L,), inv1), jnp.full((L,), zp1)]).astype(jnp.float32)

    rowsum, emin = _make_pass_b(shape, NW, L)(inputs, rowmax, lut16, scal1)
    outmn = jnp.min(emin / rowsum)
    outmx = jnp.max(1.0 / rowsum)
    scale2 = jnp.maximum((outmx - outmn) / 255.0, jnp.float32(1e-12))
    zp2 = jnp.clip(jnp.round(-128.0 - outmn / scale2), -128, 127)
    scal2 = jnp.concatenate(
        [jnp.full((L,), inv1), jnp.full((L,), zp1),
         jnp.full((L,), 1.0 / scale2),
         jnp.full((L,), zp2 + jnp.float32(_MAGIC)),
         jnp.full((L,), scale2)]).astype(jnp.float32)

    return _make_pass_c(shape, NW, L // 2)(inputs, rowmax, 1.0 / rowsum,
                                           lut16, scal2)


# speedup vs baseline: 13.7204x; 1.0172x over previous
import functools

import jax
import jax.numpy as jnp
import numpy as np
from jax import lax
from jax.experimental import pallas as pl
from jax.experimental.pallas import tpu as pltpu
from jax.experimental.pallas import tpu_sc as plsc

_MAGIC = np.float32(12582912.0)
_KMAGIC = 1262485504
_KME = _KMAGIC - 384
L = 16
U = 4


def _mesh():
    return plsc.VectorSubcoreMesh(core_axis_name="c", subcore_axis_name="s")


def _wid():
    info = plsc.get_sparse_core_info()
    return lax.axis_index("s") * info.num_cores + lax.axis_index("c")


def _tree(fn, vs):
    vs = list(vs)
    while len(vs) > 1:
        vs = [fn(vs[i], vs[i + 1]) for i in range(0, len(vs) - 1, 2)] + (
            [vs[-1]] if len(vs) % 2 else [])
    return vs[0]


def _lane_iota():
    return lax.iota(jnp.int32, L)


def _code16(v, inv1, off, lane):
    t = v * inv1 + off
    iu = lax.bitcast_convert_type(t + _MAGIC, jnp.int32)
    return jnp.left_shift(iu - _KME, 4) + lane


def _splat(ref, i):
    return plsc.load_gather(ref, [jnp.full((L,), 0, jnp.int32) + i])


def _hs(row, sh, mask):
    s = jnp.bitwise_and(row, mask)
    return jnp.right_shift(row, sh), pl.multiple_of(s, 8)


def _make_pass_a(shape, NW, BLKR):
    _, H, S, C = shape
    R = H * S
    rows_w = R // NW
    nblk = rows_w // BLKR
    sh, mask = (S - 1).bit_length(), S - 1

    @functools.partial(
        pl.kernel,
        out_type=(
            jax.ShapeDtypeStruct((R,), jnp.float32),
            jax.ShapeDtypeStruct((NW * L,), jnp.float32),
        ),
        mesh=_mesh(),
        compiler_params=pltpu.CompilerParams(needs_layout_passes=False),
        scratch_types=[
            pltpu.VMEM((BLKR, C), jnp.float32),
            pltpu.VMEM((BLKR, C), jnp.float32),
            pltpu.VMEM((BLKR, C), jnp.float32),
            pltpu.VMEM((rows_w,), jnp.float32),
            pltpu.VMEM((L,), jnp.float32),
            pltpu.SemaphoreType.DMA,
            pltpu.SemaphoreType.DMA,
            pltpu.SemaphoreType.DMA,
        ],
    )
    def pass_a(x4, rowmax_hbm, gpart_hbm, xb0, xb1, xb2, rmv, gv,
               s0, s1, s2):
        wid = _wid()
        base = wid * rows_w
        lane = _lane_iota()

        def fill(b, buf, sem):
            h, s = _hs(base + b * BLKR, sh, mask)
            pltpu.async_copy(x4.at[0, h, pl.ds(s, BLKR), :], buf, sem)

        def wait_fill(buf, sem):
            pltpu.make_async_copy(
                x4.at[0, 0, pl.ds(0, BLKR), :], buf, sem).wait()

        def process(b, buf, g):
            def row_body(r, carry):
                g, rmblk = carry

                def cb_body(cb, acc):
                    mx, mn = acc
                    o = cb * (U * L)
                    vs = [buf[r, pl.ds(o + u * L, L)] for u in range(U)]
                    return (jnp.maximum(mx, _tree(jnp.maximum, vs)),
                            jnp.minimum(mn, _tree(jnp.minimum, vs)))

                mxv, mnv = plsc.parallel_loop(
                    0, C // (U * L), 1, unroll=2,
                    carry=(jnp.full((L,), -jnp.inf, jnp.float32),
                           jnp.full((L,), jnp.inf, jnp.float32)),
                )(cb_body)
                rmax = jnp.max(mxv)
                rmin = jnp.min(mnv)
                rmblk = jnp.where(lane == r, jnp.full((L,), rmax), rmblk)
                return (jnp.minimum(g, rmin - rmax), rmblk)

            g, rmblk = lax.fori_loop(
                0, BLKR, row_body, (g, jnp.zeros((L,), jnp.float32)))
            rmv[pl.ds(b * BLKR, L)] = rmblk
            return g

        fill(0, xb0, s0)
        fill(1, xb1, s1)
        fill(2, xb2, s2)

        def outer(gi, g):
            b0 = gi * 3
            for p, (buf, sem) in enumerate(
                    ((xb0, s0), (xb1, s1), (xb2, s2))):
                b = b0 + p
                wait_fill(buf, sem)
                g = process(b, buf, g)

                @pl.when(b + 3 < nblk)
                def _():
                    fill(b + 3, buf, sem)

            return g

        g = lax.fori_loop(0, nblk // 3, outer, jnp.float32(jnp.inf))
        gv[...] = jnp.full((L,), g)
        pltpu.sync_copy(rmv, rowmax_hbm.at[pl.ds(base, rows_w)])
        pltpu.sync_copy(gv, gpart_hbm.at[pl.ds(wid * L, L)])

    return pass_a


def _make_pass_b(shape, NW, BLKR):
    _, H, S, C = shape
    R = H * S
    rows_w = R // NW
    nblk = rows_w // BLKR
    sh, mask = (S - 1).bit_length(), S - 1

    @functools.partial(
        pl.kernel,
        out_type=(
            jax.ShapeDtypeStruct((R,), jnp.float32),
            jax.ShapeDtypeStruct((R,), jnp.float32),
        ),
        mesh=_mesh(),
        compiler_params=pltpu.CompilerParams(needs_layout_passes=False),
        scratch_types=[
            pltpu.VMEM((BLKR, C), jnp.float32),
            pltpu.VMEM((BLKR, C), jnp.float32),
            pltpu.VMEM((rows_w,), jnp.float32),
            pltpu.VMEM((rows_w,), jnp.float32),
            pltpu.VMEM((rows_w,), jnp.float32),
            pltpu.VMEM((512 * L,), jnp.float32),
            pltpu.VMEM((2 * L,), jnp.float32),
            pltpu.SemaphoreType.DMA,
            pltpu.SemaphoreType.DMA,
        ],
    )
    def pass_b(x4, rowmax_hbm, lut_hbm, scal_hbm, rowsum_hbm, emin_hbm,
               xb0, xb1, rmv, rsv, emv, lutv, scv, s0, s1):
        wid = _wid()
        base = wid * rows_w
        lane = _lane_iota()
        pltpu.sync_copy(lut_hbm, lutv)
        pltpu.sync_copy(scal_hbm, scv)
        pltpu.sync_copy(rowmax_hbm.at[pl.ds(base, rows_w)], rmv)
        inv1 = scv[pl.ds(0, L)]
        zp1 = scv[pl.ds(L, L)]

        def fill(b, buf, sem):
            h, s = _hs(base + b * BLKR, sh, mask)
            pltpu.async_copy(x4.at[0, h, pl.ds(s, BLKR), :], buf, sem)

        def wait_fill(buf, sem):
            pltpu.make_async_copy(
                x4.at[0, 0, pl.ds(0, BLKR), :], buf, sem).wait()

        def process(b, buf):
            def row_body(r, carry):
                rsblk, emblk = carry
                rm = _splat(rmv, b * BLKR + r)
                off = zp1 - rm * inv1

                def cb_body(cb, acc):
                    sacc, macc = acc
                    o = cb * (U * L)
                    es = []
                    for u in range(U):
                        v = buf[r, pl.ds(o + u * L, L)]
                        es.append(
                            plsc.load_gather(lutv,
                                             [_code16(v, inv1, off, lane)]))
                    return (sacc + _tree(jnp.add, es),
                            jnp.minimum(macc, _tree(jnp.minimum, es)))

                sacc, macc = plsc.parallel_loop(
                    0, C // (U * L), 1, unroll=4,
                    carry=(jnp.zeros((L,), jnp.float32),
                           jnp.full((L,), jnp.inf, jnp.float32)),
                )(cb_body)
                rsum = jnp.sum(sacc)
                rmin = jnp.min(macc)
                rsblk = jnp.where(lane == r, jnp.full((L,), rsum), rsblk)
                emblk = jnp.where(lane == r, jnp.full((L,), rmin), emblk)
                return (rsblk, emblk)

            rsblk, emblk = lax.fori_loop(
                0, BLKR, row_body,
                (jnp.zeros((L,), jnp.float32), jnp.zeros((L,), jnp.float32)))
            rsv[pl.ds(b * BLKR, L)] = rsblk
            emv[pl.ds(b * BLKR, L)] = emblk

        fill(0, xb0, s0)
        fill(1, xb1, s1)

        def outer(gi, _):
            b0 = gi * 2

            wait_fill(xb0, s0)
            process(b0, xb0)

            @pl.when(b0 + 2 < nblk)
            def _():
                fill(b0 + 2, xb0, s0)

            wait_fill(xb1, s1)
            process(b0 + 1, xb1)

            @pl.when(b0 + 3 < nblk)
            def _():
                fill(b0 + 3, xb1, s1)

            return 0

        lax.fori_loop(0, nblk // 2, outer, 0)
        pltpu.sync_copy(rsv, rowsum_hbm.at[pl.ds(base, rows_w)])
        pltpu.sync_copy(emv, emin_hbm.at[pl.ds(base, rows_w)])

    return pass_b


def _make_pass_c(shape, NW, BLKR):
    _, H, S, C = shape
    R = H * S
    rows_w = R // NW
    nblk = rows_w // BLKR
    sh, mask = (S - 1).bit_length(), S - 1

    @functools.partial(
        pl.kernel,
        out_type=jax.ShapeDtypeStruct(shape, jnp.float32),
        mesh=_mesh(),
        compiler_params=pltpu.CompilerParams(needs_layout_passes=False),
        scratch_types=[
            pltpu.VMEM((BLKR, C), jnp.float32),
            pltpu.VMEM((BLKR, C), jnp.float32),
            pltpu.VMEM((BLKR, C), jnp.float32),
            pltpu.VMEM((BLKR, C), jnp.float32),
            pltpu.VMEM((rows_w,), jnp.float32),
            pltpu.VMEM((rows_w,), jnp.float32),
            pltpu.VMEM((512 * L,), jnp.float32),
            pltpu.VMEM((5 * L,), jnp.float32),
            pltpu.SemaphoreType.DMA,
            pltpu.SemaphoreType.DMA,
            pltpu.SemaphoreType.DMA,
            pltpu.SemaphoreType.DMA,
        ],
    )
    def pass_c(x4, rowmax_hbm, invsum_hbm, lut_hbm, scal_hbm, out4,
               xb0, xb1, ob0, ob1, rmv, rsv, lutv, scv, si0, si1, so0, so1):
        wid = _wid()
        base = wid * rows_w
        lane = _lane_iota()
        pltpu.sync_copy(lut_hbm, lutv)
        pltpu.sync_copy(scal_hbm, scv)
        pltpu.sync_copy(rowmax_hbm.at[pl.ds(base, rows_w)], rmv)
        pltpu.sync_copy(invsum_hbm.at[pl.ds(base, rows_w)], rsv)
        inv1 = scv[pl.ds(0, L)]
        zp1 = scv[pl.ds(L, L)]
        inv2 = scv[pl.ds(2 * L, L)]
        zp2m = scv[pl.ds(3 * L, L)]
        s2 = scv[pl.ds(4 * L, L)]
        hi = np.float32(12582912.0 + 127.0)

        def fill(b, buf, sem):
            h, s = _hs(base + b * BLKR, sh, mask)
            pltpu.async_copy(x4.at[0, h, pl.ds(s, BLKR), :], buf, sem)

        def wait_fill(buf, sem):
            pltpu.make_async_copy(
                x4.at[0, 0, pl.ds(0, BLKR), :], buf, sem).wait()

        def drain(b, buf, sem):
            h, s = _hs(base + b * BLKR, sh, mask)
            pltpu.async_copy(buf, out4.at[0, h, pl.ds(s, BLKR), :], sem)

        def wait_drain(buf, sem):
            pltpu.make_async_copy(
                buf, out4.at[0, 0, pl.ds(0, BLKR), :], sem).wait()

        def process(b, buf, obuf):
            def row_body(r, _):
                rm = _splat(rmv, b * BLKR + r)
                rs = _splat(rsv, b * BLKR + r)
                off = zp1 - rm * inv1
                k2 = rs * inv2

                def cb_body(cb):
                    o = cb * (U * L)
                    for u in range(U):
                        v = buf[r, pl.ds(o + u * L, L)]
                        e = plsc.load_gather(
                            lutv, [_code16(v, inv1, off, lane)])
                        t2m = e * k2 + zp2m
                        q2m = jnp.minimum(t2m, hi)
                        obuf[r, pl.ds(o + u * L, L)] = (q2m - zp2m) * s2

                plsc.parallel_loop(0, C // (U * L), 1, unroll=4)(cb_body)
                return 0

            lax.fori_loop(0, BLKR, row_body, 0)

        fill(0, xb0, si0)
        fill(1, xb1, si1)

        def outer(gi, _):
            b0 = gi * 2

            wait_fill(xb0, si0)

            @pl.when(b0 >= 2)
            def _():
                wait_drain(ob0, so0)

            process(b0, xb0, ob0)
            drain(b0, ob0, so0)

            @pl.when(b0 + 2 < nblk)
            def _():
                fill(b0 + 2, xb0, si0)

            wait_fill(xb1, si1)

            @pl.when(b0 >= 2)
            def _():
                wait_drain(ob1, so1)

            process(b0 + 1, xb1, ob1)
            drain(b0 + 1, ob1, so1)

            @pl.when(b0 + 3 < nblk)
            def _():
                fill(b0 + 3, xb1, si1)

            return 0

        lax.fori_loop(0, nblk // 2, outer, 0)
        wait_drain(ob0, so0)
        wait_drain(ob1, so1)

    return pass_c


def kernel(inputs):
    shape = inputs.shape
    NW = 32

    rowmax, gpart = _make_pass_a(shape, NW, L)(inputs)
    mn = jnp.min(gpart)
    mx = jnp.float32(0.0)
    scale1 = jnp.maximum((mx - mn) / 255.0, jnp.float32(1e-12))
    zp1 = jnp.clip(jnp.round(-128.0 - mn / scale1), -128, 127)
    codes = jnp.clip(jnp.arange(512, dtype=jnp.float32) - 256.0, 0.0, 255.0)
    lut = jnp.exp(((codes - 128.0) - zp1) * scale1)
    lut16 = jnp.repeat(lut, L)
    inv1 = 1.0 / scale1
    scal1 = jnp.concatenate(
        [jnp.full((L,), inv1), jnp.full((L,), zp1)]).astype(jnp.float32)

    rowsum, emin = _make_pass_b(shape, NW, L)(inputs, rowmax, lut16, scal1)
    outmn = jnp.min(emin / rowsum)
    outmx = jnp.max(1.0 / rowsum)
    scale2 = jnp.maximum((outmx - outmn) / 255.0, jnp.float32(1e-12))
    zp2 = jnp.clip(jnp.round(-128.0 - outmn / scale2), -128, 127)
    scal2 = jnp.concatenate(
        [jnp.full((L,), inv1), jnp.full((L,), zp1),
         jnp.full((L,), 1.0 / scale2),
         jnp.full((L,), zp2 + jnp.float32(_MAGIC)),
         jnp.full((L,), scale2)]).astype(jnp.float32)

    return _make_pass_c(shape, NW, L // 2)(inputs, rowmax, 1.0 / rowsum,
                                           lut16, scal2)


# speedup vs baseline: 13.9713x; 1.0183x over previous
import functools

import jax
import jax.numpy as jnp
import numpy as np
from jax import lax
from jax.experimental import pallas as pl
from jax.experimental.pallas import tpu as pltpu
from jax.experimental.pallas import tpu_sc as plsc

_MAGIC = np.float32(12582912.0)
_KMAGIC = 1262485504
_KME = _KMAGIC - 384
L = 16
U = 4


def _mesh():
    return plsc.VectorSubcoreMesh(core_axis_name="c", subcore_axis_name="s")


def _wid():
    info = plsc.get_sparse_core_info()
    return lax.axis_index("s") * info.num_cores + lax.axis_index("c")


def _tree(fn, vs):
    vs = list(vs)
    while len(vs) > 1:
        vs = [fn(vs[i], vs[i + 1]) for i in range(0, len(vs) - 1, 2)] + (
            [vs[-1]] if len(vs) % 2 else [])
    return vs[0]


def _lane_iota():
    return lax.iota(jnp.int32, L)


def _code16(v, inv1, off, lane):
    t = v * inv1 + off
    iu = lax.bitcast_convert_type(t + _MAGIC, jnp.int32)
    return jnp.left_shift(iu - _KME, 4) + lane


def _splat(ref, i):
    return plsc.load_gather(ref, [jnp.full((L,), 0, jnp.int32) + i])


def _hs(row, sh, mask):
    s = jnp.bitwise_and(row, mask)
    return jnp.right_shift(row, sh), pl.multiple_of(s, 8)


def _make_pass_a(shape, NW, BLKR):
    _, H, S, C = shape
    R = H * S
    rows_w = R // NW
    nblk = rows_w // BLKR
    sh, mask = (S - 1).bit_length(), S - 1

    @functools.partial(
        pl.kernel,
        out_type=(
            jax.ShapeDtypeStruct((R,), jnp.float32),
            jax.ShapeDtypeStruct((NW * L,), jnp.float32),
        ),
        mesh=_mesh(),
        compiler_params=pltpu.CompilerParams(needs_layout_passes=False),
        scratch_types=[
            pltpu.VMEM((BLKR, C), jnp.float32),
            pltpu.VMEM((BLKR, C), jnp.float32),
            pltpu.VMEM((BLKR, C), jnp.float32),
            pltpu.VMEM((rows_w,), jnp.float32),
            pltpu.VMEM((L,), jnp.float32),
            pltpu.SemaphoreType.DMA,
            pltpu.SemaphoreType.DMA,
            pltpu.SemaphoreType.DMA,
        ],
    )
    def pass_a(x4, rowmax_hbm, gpart_hbm, xb0, xb1, xb2, rmv, gv,
               s0, s1, s2):
        wid = _wid()
        base = wid * rows_w
        lane = _lane_iota()

        def fill(b, buf, sem):
            h, s = _hs(base + b * BLKR, sh, mask)
            pltpu.async_copy(x4.at[0, h, pl.ds(s, BLKR), :], buf, sem)

        def wait_fill(buf, sem):
            pltpu.make_async_copy(
                x4.at[0, 0, pl.ds(0, BLKR), :], buf, sem).wait()

        def process(b, buf, g):
            def row_body(r, carry):
                g, rmblk = carry

                def cb_body(cb, acc):
                    mx, mn = acc
                    o = cb * (U * L)
                    vs = [buf[r, pl.ds(o + u * L, L)] for u in range(U)]
                    return (jnp.maximum(mx, _tree(jnp.maximum, vs)),
                            jnp.minimum(mn, _tree(jnp.minimum, vs)))

                mxv, mnv = plsc.parallel_loop(
                    0, C // (U * L), 1, unroll=4,
                    carry=(jnp.full((L,), -jnp.inf, jnp.float32),
                           jnp.full((L,), jnp.inf, jnp.float32)),
                )(cb_body)
                rmax = jnp.max(mxv)
                rmin = jnp.min(mnv)
                rmblk = jnp.where(lane == r, jnp.full((L,), rmax), rmblk)
                return (jnp.minimum(g, rmin - rmax), rmblk)

            g, rmblk = lax.fori_loop(
                0, BLKR, row_body, (g, jnp.zeros((L,), jnp.float32)))
            rmv[pl.ds(b * BLKR, L)] = rmblk
            return g

        fill(0, xb0, s0)
        fill(1, xb1, s1)
        fill(2, xb2, s2)

        def outer(gi, g):
            b0 = gi * 3
            for p, (buf, sem) in enumerate(
                    ((xb0, s0), (xb1, s1), (xb2, s2))):
                b = b0 + p
                wait_fill(buf, sem)
                g = process(b, buf, g)

                @pl.when(b + 3 < nblk)
                def _():
                    fill(b + 3, buf, sem)

            return g

        g = lax.fori_loop(0, nblk // 3, outer, jnp.float32(jnp.inf))
        gv[...] = jnp.full((L,), g)
        pltpu.sync_copy(rmv, rowmax_hbm.at[pl.ds(base, rows_w)])
        pltpu.sync_copy(gv, gpart_hbm.at[pl.ds(wid * L, L)])

    return pass_a


def _make_pass_b(shape, NW, BLKR):
    _, H, S, C = shape
    R = H * S
    rows_w = R // NW
    nblk = rows_w // BLKR
    sh, mask = (S - 1).bit_length(), S - 1

    @functools.partial(
        pl.kernel,
        out_type=(
            jax.ShapeDtypeStruct((R,), jnp.float32),
            jax.ShapeDtypeStruct((R,), jnp.float32),
        ),
        mesh=_mesh(),
        compiler_params=pltpu.CompilerParams(needs_layout_passes=False),
        scratch_types=[
            pltpu.VMEM((BLKR, C), jnp.float32),
            pltpu.VMEM((BLKR, C), jnp.float32),
            pltpu.VMEM((rows_w,), jnp.float32),
            pltpu.VMEM((rows_w,), jnp.float32),
            pltpu.VMEM((rows_w,), jnp.float32),
            pltpu.VMEM((512 * L,), jnp.float32),
            pltpu.VMEM((2 * L,), jnp.float32),
            pltpu.SemaphoreType.DMA,
            pltpu.SemaphoreType.DMA,
        ],
    )
    def pass_b(x4, rowmax_hbm, lut_hbm, scal_hbm, rowsum_hbm, emin_hbm,
               xb0, xb1, rmv, rsv, emv, lutv, scv, s0, s1):
        wid = _wid()
        base = wid * rows_w
        lane = _lane_iota()
        pltpu.sync_copy(lut_hbm, lutv)
        pltpu.sync_copy(scal_hbm, scv)
        pltpu.sync_copy(rowmax_hbm.at[pl.ds(base, rows_w)], rmv)
        inv1 = scv[pl.ds(0, L)]
        zp1 = scv[pl.ds(L, L)]

        def fill(b, buf, sem):
            h, s = _hs(base + b * BLKR, sh, mask)
            pltpu.async_copy(x4.at[0, h, pl.ds(s, BLKR), :], buf, sem)

        def wait_fill(buf, sem):
            pltpu.make_async_copy(
                x4.at[0, 0, pl.ds(0, BLKR), :], buf, sem).wait()

        def process(b, buf):
            def row_body(r, carry):
                rsblk, emblk = carry
                rm = _splat(rmv, b * BLKR + r)
                off = zp1 - rm * inv1

                def cb_body(cb, acc):
                    sacc, macc = acc
                    o = cb * (U * L)
                    es = []
                    for u in range(U):
                        v = buf[r, pl.ds(o + u * L, L)]
                        es.append(
                            plsc.load_gather(lutv,
                                             [_code16(v, inv1, off, lane)]))
                    return (sacc + _tree(jnp.add, es),
                            jnp.minimum(macc, _tree(jnp.minimum, es)))

                sacc, macc = plsc.parallel_loop(
                    0, C // (U * L), 1, unroll=2,
                    carry=(jnp.zeros((L,), jnp.float32),
                           jnp.full((L,), jnp.inf, jnp.float32)),
                )(cb_body)
                rsum = jnp.sum(sacc)
                rmin = jnp.min(macc)
                rsblk = jnp.where(lane == r, jnp.full((L,), rsum), rsblk)
                emblk = jnp.where(lane == r, jnp.full((L,), rmin), emblk)
                return (rsblk, emblk)

            rsblk, emblk = lax.fori_loop(
                0, BLKR, row_body,
                (jnp.zeros((L,), jnp.float32), jnp.zeros((L,), jnp.float32)))
            rsv[pl.ds(b * BLKR, L)] = rsblk
            emv[pl.ds(b * BLKR, L)] = emblk

        fill(0, xb0, s0)
        fill(1, xb1, s1)

        def outer(gi, _):
            b0 = gi * 2

            wait_fill(xb0, s0)
            process(b0, xb0)

            @pl.when(b0 + 2 < nblk)
            def _():
                fill(b0 + 2, xb0, s0)

            wait_fill(xb1, s1)
            process(b0 + 1, xb1)

            @pl.when(b0 + 3 < nblk)
            def _():
                fill(b0 + 3, xb1, s1)

            return 0

        lax.fori_loop(0, nblk // 2, outer, 0)
        pltpu.sync_copy(rsv, rowsum_hbm.at[pl.ds(base, rows_w)])
        pltpu.sync_copy(emv, emin_hbm.at[pl.ds(base, rows_w)])

    return pass_b


def _make_pass_c(shape, NW, BLKR):
    _, H, S, C = shape
    R = H * S
    rows_w = R // NW
    nblk = rows_w // BLKR
    sh, mask = (S - 1).bit_length(), S - 1

    @functools.partial(
        pl.kernel,
        out_type=jax.ShapeDtypeStruct(shape, jnp.float32),
        mesh=_mesh(),
        compiler_params=pltpu.CompilerParams(needs_layout_passes=False),
        scratch_types=[
            pltpu.VMEM((BLKR, C), jnp.float32),
            pltpu.VMEM((BLKR, C), jnp.float32),
            pltpu.VMEM((BLKR, C), jnp.float32),
            pltpu.VMEM((BLKR, C), jnp.float32),
            pltpu.VMEM((rows_w,), jnp.float32),
            pltpu.VMEM((rows_w,), jnp.float32),
            pltpu.VMEM((512 * L,), jnp.float32),
            pltpu.VMEM((5 * L,), jnp.float32),
            pltpu.SemaphoreType.DMA,
            pltpu.SemaphoreType.DMA,
            pltpu.SemaphoreType.DMA,
            pltpu.SemaphoreType.DMA,
        ],
    )
    def pass_c(x4, rowmax_hbm, invsum_hbm, lut_hbm, scal_hbm, out4,
               xb0, xb1, ob0, ob1, rmv, rsv, lutv, scv, si0, si1, so0, so1):
        wid = _wid()
        base = wid * rows_w
        lane = _lane_iota()
        pltpu.sync_copy(lut_hbm, lutv)
        pltpu.sync_copy(scal_hbm, scv)
        pltpu.sync_copy(rowmax_hbm.at[pl.ds(base, rows_w)], rmv)
        pltpu.sync_copy(invsum_hbm.at[pl.ds(base, rows_w)], rsv)
        inv1 = scv[pl.ds(0, L)]
        zp1 = scv[pl.ds(L, L)]
        inv2 = scv[pl.ds(2 * L, L)]
        zp2m = scv[pl.ds(3 * L, L)]
        s2 = scv[pl.ds(4 * L, L)]
        hi = np.float32(12582912.0 + 127.0)

        def fill(b, buf, sem):
            h, s = _hs(base + b * BLKR, sh, mask)
            pltpu.async_copy(x4.at[0, h, pl.ds(s, BLKR), :], buf, sem)

        def wait_fill(buf, sem):
            pltpu.make_async_copy(
                x4.at[0, 0, pl.ds(0, BLKR), :], buf, sem).wait()

        def drain(b, buf, sem):
            h, s = _hs(base + b * BLKR, sh, mask)
            pltpu.async_copy(buf, out4.at[0, h, pl.ds(s, BLKR), :], sem)

        def wait_drain(buf, sem):
            pltpu.make_async_copy(
                buf, out4.at[0, 0, pl.ds(0, BLKR), :], sem).wait()

        def process(b, buf, obuf):
            def row_body(r, _):
                rm = _splat(rmv, b * BLKR + r)
                rs = _splat(rsv, b * BLKR + r)
                off = zp1 - rm * inv1
                k2 = rs * inv2

                def cb_body(cb):
                    o = cb * (U * L)
                    for u in range(U):
                        v = buf[r, pl.ds(o + u * L, L)]
                        e = plsc.load_gather(
                            lutv, [_code16(v, inv1, off, lane)])
                        t2m = e * k2 + zp2m
                        q2m = jnp.minimum(t2m, hi)
                        obuf[r, pl.ds(o + u * L, L)] = (q2m - zp2m) * s2

                plsc.parallel_loop(0, C // (U * L), 1, unroll=4)(cb_body)
                return 0

            lax.fori_loop(0, BLKR, row_body, 0)

        fill(0, xb0, si0)
        fill(1, xb1, si1)

        def outer(gi, _):
            b0 = gi * 2

            wait_fill(xb0, si0)

            @pl.when(b0 >= 2)
            def _():
                wait_drain(ob0, so0)

            process(b0, xb0, ob0)
            drain(b0, ob0, so0)

            @pl.when(b0 + 2 < nblk)
            def _():
                fill(b0 + 2, xb0, si0)

            wait_fill(xb1, si1)

            @pl.when(b0 >= 2)
            def _():
                wait_drain(ob1, so1)

            process(b0 + 1, xb1, ob1)
            drain(b0 + 1, ob1, so1)

            @pl.when(b0 + 3 < nblk)
            def _():
                fill(b0 + 3, xb1, si1)

            return 0

        lax.fori_loop(0, nblk // 2, outer, 0)
        wait_drain(ob0, so0)
        wait_drain(ob1, so1)

    return pass_c


def kernel(inputs):
    shape = inputs.shape
    NW = 32

    rowmax, gpart = _make_pass_a(shape, NW, L)(inputs)
    mn = jnp.min(gpart)
    mx = jnp.float32(0.0)
    scale1 = jnp.maximum((mx - mn) / 255.0, jnp.float32(1e-12))
    zp1 = jnp.clip(jnp.round(-128.0 - mn / scale1), -128, 127)
    codes = jnp.clip(jnp.arange(512, dtype=jnp.float32) - 256.0, 0.0, 255.0)
    lut = jnp.exp(((codes - 128.0) - zp1) * scale1)
    lut16 = jnp.repeat(lut, L)
    inv1 = 1.0 / scale1
    scal1 = jnp.concatenate(
        [jnp.full((L,), inv1), jnp.full((L,), zp1)]).astype(jnp.float32)

    rowsum, emin = _make_pass_b(shape, NW, L)(inputs, rowmax, lut16, scal1)
    outmn = jnp.min(emin / rowsum)
    outmx = jnp.max(1.0 / rowsum)
    scale2 = jnp.maximum((outmx - outmn) / 255.0, jnp.float32(1e-12))
    zp2 = jnp.clip(jnp.round(-128.0 - outmn / scale2), -128, 127)
    scal2 = jnp.concatenate(
        [jnp.full((L,), inv1), jnp.full((L,), zp1),
         jnp.full((L,), 1.0 / scale2),
         jnp.full((L,), zp2 + jnp.float32(_MAGIC)),
         jnp.full((L,), scale2)]).astype(jnp.float32)

    return _make_pass_c(shape, NW, L // 2)(inputs, rowmax, 1.0 / rowsum,
                                           lut16, scal2)


# speedup vs baseline: 14.0159x; 1.0032x over previous
import functools

import jax
import jax.numpy as jnp
import numpy as np
from jax import lax
from jax.experimental import pallas as pl
from jax.experimental.pallas import tpu as pltpu
from jax.experimental.pallas import tpu_sc as plsc

_MAGIC = np.float32(12582912.0)
_KMAGIC = 1262485504
_KME = _KMAGIC - 384
L = 16
U = 4


def _mesh():
    return plsc.VectorSubcoreMesh(core_axis_name="c", subcore_axis_name="s")


def _wid():
    info = plsc.get_sparse_core_info()
    return lax.axis_index("s") * info.num_cores + lax.axis_index("c")


def _tree(fn, vs):
    vs = list(vs)
    while len(vs) > 1:
        vs = [fn(vs[i], vs[i + 1]) for i in range(0, len(vs) - 1, 2)] + (
            [vs[-1]] if len(vs) % 2 else [])
    return vs[0]


def _lane_iota():
    return lax.iota(jnp.int32, L)


def _code16(v, inv1, off, lane):
    t = v * inv1 + off
    iu = lax.bitcast_convert_type(t + _MAGIC, jnp.int32)
    return jnp.left_shift(iu - _KME, 4) + lane


def _splat(ref, i):
    return plsc.load_gather(ref, [jnp.full((L,), 0, jnp.int32) + i])


def _hs(row, sh, mask):
    s = jnp.bitwise_and(row, mask)
    return jnp.right_shift(row, sh), pl.multiple_of(s, 8)


def _make_pass_a(shape, NW, BLKR):
    _, H, S, C = shape
    R = H * S
    rows_w = R // NW
    nblk = rows_w // BLKR
    sh, mask = (S - 1).bit_length(), S - 1

    @functools.partial(
        pl.kernel,
        out_type=(
            jax.ShapeDtypeStruct((R,), jnp.float32),
            jax.ShapeDtypeStruct((NW * L,), jnp.float32),
        ),
        mesh=_mesh(),
        compiler_params=pltpu.CompilerParams(needs_layout_passes=False),
        scratch_types=[
            pltpu.VMEM((BLKR, C), jnp.float32),
            pltpu.VMEM((BLKR, C), jnp.float32),
            pltpu.VMEM((BLKR, C), jnp.float32),
            pltpu.VMEM((rows_w,), jnp.float32),
            pltpu.VMEM((L,), jnp.float32),
            pltpu.SemaphoreType.DMA,
            pltpu.SemaphoreType.DMA,
            pltpu.SemaphoreType.DMA,
        ],
    )
    def pass_a(x4, rowmax_hbm, gpart_hbm, xb0, xb1, xb2, rmv, gv,
               s0, s1, s2):
        wid = _wid()
        base = wid * rows_w
        lane = _lane_iota()

        def fill(b, buf, sem):
            h, s = _hs(base + b * BLKR, sh, mask)
            pltpu.async_copy(x4.at[0, h, pl.ds(s, BLKR), :], buf, sem)

        def wait_fill(buf, sem):
            pltpu.make_async_copy(
                x4.at[0, 0, pl.ds(0, BLKR), :], buf, sem).wait()

        def process(b, buf, g):
            def row_body(r, carry):
                g, rmblk = carry

                def cb_body(cb, acc):
                    mx, mn = acc
                    o = cb * (U * L)
                    vs = [buf[r, pl.ds(o + u * L, L)] for u in range(U)]
                    return (jnp.maximum(mx, _tree(jnp.maximum, vs)),
                            jnp.minimum(mn, _tree(jnp.minimum, vs)))

                mxv, mnv = plsc.parallel_loop(
                    0, C // (U * L), 1, unroll=4,
                    carry=(jnp.full((L,), -jnp.inf, jnp.float32),
                           jnp.full((L,), jnp.inf, jnp.float32)),
                )(cb_body)
                rmax = jnp.max(mxv)
                rmin = jnp.min(mnv)
                rmblk = jnp.where(lane == r, jnp.full((L,), rmax), rmblk)
                return (jnp.minimum(g, rmin - rmax), rmblk)

            g, rmblk = lax.fori_loop(
                0, BLKR, row_body, (g, jnp.zeros((L,), jnp.float32)))
            rmv[pl.ds(b * BLKR, L)] = rmblk
            return g

        fill(0, xb0, s0)
        fill(1, xb1, s1)
        fill(2, xb2, s2)

        def outer(gi, g):
            b0 = gi * 3
            for p, (buf, sem) in enumerate(
                    ((xb0, s0), (xb1, s1), (xb2, s2))):
                b = b0 + p
                wait_fill(buf, sem)
                g = process(b, buf, g)

                @pl.when(b + 3 < nblk)
                def _():
                    fill(b + 3, buf, sem)

            return g

        g = lax.fori_loop(0, nblk // 3, outer, jnp.float32(jnp.inf))
        gv[...] = jnp.full((L,), g)
        pltpu.sync_copy(rmv, rowmax_hbm.at[pl.ds(base, rows_w)])
        pltpu.sync_copy(gv, gpart_hbm.at[pl.ds(wid * L, L)])

    return pass_a


def _make_pass_b(shape, NW, BLKR):
    _, H, S, C = shape
    R = H * S
    rows_w = R // NW
    nblk = rows_w // BLKR
    sh, mask = (S - 1).bit_length(), S - 1

    @functools.partial(
        pl.kernel,
        out_type=(
            jax.ShapeDtypeStruct((R,), jnp.float32),
            jax.ShapeDtypeStruct((NW * L,), jnp.float32),
            jax.ShapeDtypeStruct((NW * L,), jnp.float32),
        ),
        mesh=_mesh(),
        compiler_params=pltpu.CompilerParams(needs_layout_passes=False),
        scratch_types=[
            pltpu.VMEM((BLKR, C), jnp.float32),
            pltpu.VMEM((BLKR, C), jnp.float32),
            pltpu.VMEM((rows_w,), jnp.float32),
            pltpu.VMEM((rows_w,), jnp.float32),
            pltpu.VMEM((rows_w,), jnp.float32),
            pltpu.VMEM((512 * L,), jnp.float32),
            pltpu.VMEM((2 * L,), jnp.float32),
            pltpu.SemaphoreType.DMA,
            pltpu.SemaphoreType.DMA,
        ],
    )
    def pass_b(x4, rowmax_hbm, lut_hbm, scal_hbm, invsum_hbm, pmn_hbm,
               pmx_hbm, xb0, xb1, rmv, rsv, emv, lutv, scv, s0, s1):
        wid = _wid()
        base = wid * rows_w
        lane = _lane_iota()
        pltpu.sync_copy(lut_hbm, lutv)
        pltpu.sync_copy(scal_hbm, scv)
        pltpu.sync_copy(rowmax_hbm.at[pl.ds(base, rows_w)], rmv)
        inv1 = scv[pl.ds(0, L)]
        zp1 = scv[pl.ds(L, L)]

        def fill(b, buf, sem):
            h, s = _hs(base + b * BLKR, sh, mask)
            pltpu.async_copy(x4.at[0, h, pl.ds(s, BLKR), :], buf, sem)

        def wait_fill(buf, sem):
            pltpu.make_async_copy(
                x4.at[0, 0, pl.ds(0, BLKR), :], buf, sem).wait()

        def process(b, buf):
            def row_body(r, carry):
                rsblk, emblk = carry
                rm = _splat(rmv, b * BLKR + r)
                off = zp1 - rm * inv1

                def cb_body(cb, acc):
                    sacc, macc = acc
                    o = cb * (U * L)
                    es = []
                    for u in range(U):
                        v = buf[r, pl.ds(o + u * L, L)]
                        es.append(
                            plsc.load_gather(lutv,
                                             [_code16(v, inv1, off, lane)]))
                    return (sacc + _tree(jnp.add, es),
                            jnp.minimum(macc, _tree(jnp.minimum, es)))

                sacc, macc = plsc.parallel_loop(
                    0, C // (U * L), 1, unroll=2,
                    carry=(jnp.zeros((L,), jnp.float32),
                           jnp.full((L,), jnp.inf, jnp.float32)),
                )(cb_body)
                rsum = jnp.sum(sacc)
                rmin = jnp.min(macc)
                rsblk = jnp.where(lane == r, jnp.full((L,), rsum), rsblk)
                emblk = jnp.where(lane == r, jnp.full((L,), rmin), emblk)
                return (rsblk, emblk)

            rsblk, emblk = lax.fori_loop(
                0, BLKR, row_body,
                (jnp.zeros((L,), jnp.float32), jnp.zeros((L,), jnp.float32)))
            rsv[pl.ds(b * BLKR, L)] = rsblk
            emv[pl.ds(b * BLKR, L)] = emblk

        fill(0, xb0, s0)
        fill(1, xb1, s1)

        def outer(gi, _):
            b0 = gi * 2

            wait_fill(xb0, s0)
            process(b0, xb0)

            @pl.when(b0 + 2 < nblk)
            def _():
                fill(b0 + 2, xb0, s0)

            wait_fill(xb1, s1)
            process(b0 + 1, xb1)

            @pl.when(b0 + 3 < nblk)
            def _():
                fill(b0 + 3, xb1, s1)

            return 0

        lax.fori_loop(0, nblk // 2, outer, 0)

        def ep_body(i, carry):
            omn, omx = carry
            rs = rsv[pl.ds(i * L, L)]
            em = emv[pl.ds(i * L, L)]
            inv = 1.0 / rs
            rsv[pl.ds(i * L, L)] = inv
            return (jnp.minimum(omn, em * inv), jnp.maximum(omx, inv))

        omn, omx = lax.fori_loop(
            0, rows_w // L, ep_body,
            (jnp.full((L,), jnp.inf, jnp.float32),
             jnp.full((L,), -jnp.inf, jnp.float32)))
        emv[pl.ds(0, L)] = omn
        emv[pl.ds(L, L)] = omx
        pltpu.sync_copy(rsv, invsum_hbm.at[pl.ds(base, rows_w)])
        pltpu.sync_copy(emv.at[pl.ds(0, L)], pmn_hbm.at[pl.ds(wid * L, L)])
        pltpu.sync_copy(emv.at[pl.ds(L, L)], pmx_hbm.at[pl.ds(wid * L, L)])

    return pass_b


def _make_pass_c(shape, NW, BLKR):
    _, H, S, C = shape
    R = H * S
    rows_w = R // NW
    nblk = rows_w // BLKR
    sh, mask = (S - 1).bit_length(), S - 1

    @functools.partial(
        pl.kernel,
        out_type=jax.ShapeDtypeStruct(shape, jnp.float32),
        mesh=_mesh(),
        compiler_params=pltpu.CompilerParams(needs_layout_passes=False),
        scratch_types=[
            pltpu.VMEM((BLKR, C), jnp.float32),
            pltpu.VMEM((BLKR, C), jnp.float32),
            pltpu.VMEM((BLKR, C), jnp.float32),
            pltpu.VMEM((BLKR, C), jnp.float32),
            pltpu.VMEM((rows_w,), jnp.float32),
            pltpu.VMEM((rows_w,), jnp.float32),
            pltpu.VMEM((512 * L,), jnp.float32),
            pltpu.VMEM((2 * L,), jnp.float32),
            pltpu.VMEM((NW * L,), jnp.float32),
            pltpu.VMEM((NW * L,), jnp.float32),
            pltpu.SemaphoreType.DMA,
            pltpu.SemaphoreType.DMA,
            pltpu.SemaphoreType.DMA,
            pltpu.SemaphoreType.DMA,
        ],
    )
    def pass_c(x4, rowmax_hbm, invsum_hbm, pmn_hbm, pmx_hbm, lut_hbm,
               scal_hbm, out4, xb0, xb1, ob0, ob1, rmv, rsv, lutv, scv,
               pmnv, pmxv, si0, si1, so0, so1):
        wid = _wid()
        base = wid * rows_w
        lane = _lane_iota()
        pltpu.sync_copy(lut_hbm, lutv)
        pltpu.sync_copy(scal_hbm, scv)
        pltpu.sync_copy(rowmax_hbm.at[pl.ds(base, rows_w)], rmv)
        pltpu.sync_copy(invsum_hbm.at[pl.ds(base, rows_w)], rsv)
        pltpu.sync_copy(pmn_hbm, pmnv)
        pltpu.sync_copy(pmx_hbm, pmxv)
        inv1 = scv[pl.ds(0, L)]
        zp1 = scv[pl.ds(L, L)]
        hi = np.float32(12582912.0 + 127.0)

        def red_body(i, carry):
            a, b2 = carry
            return (jnp.minimum(a, pmnv[pl.ds(i * L, L)]),
                    jnp.maximum(b2, pmxv[pl.ds(i * L, L)]))

        omnv, omxv = lax.fori_loop(
            0, NW, red_body,
            (jnp.full((L,), jnp.inf, jnp.float32),
             jnp.full((L,), -jnp.inf, jnp.float32)))
        omn = jnp.full((L,), jnp.min(omnv))
        omx = jnp.full((L,), jnp.max(omxv))
        s2 = jnp.maximum((omx - omn) / 255.0, jnp.float32(1e-12))
        zp2 = jnp.clip(((-128.0 - omn / s2) + _MAGIC) - _MAGIC,
                       -128.0, 127.0)
        inv2 = 1.0 / s2
        zp2m = zp2 + _MAGIC

        def fill(b, buf, sem):
            h, s = _hs(base + b * BLKR, sh, mask)
            pltpu.async_copy(x4.at[0, h, pl.ds(s, BLKR), :], buf, sem)

        def wait_fill(buf, sem):
            pltpu.make_async_copy(
                x4.at[0, 0, pl.ds(0, BLKR), :], buf, sem).wait()

        def drain(b, buf, sem):
            h, s = _hs(base + b * BLKR, sh, mask)
            pltpu.async_copy(buf, out4.at[0, h, pl.ds(s, BLKR), :], sem)

        def wait_drain(buf, sem):
            pltpu.make_async_copy(
                buf, out4.at[0, 0, pl.ds(0, BLKR), :], sem).wait()

        def process(b, buf, obuf):
            def row_body(r, _):
                rm = _splat(rmv, b * BLKR + r)
                rs = _splat(rsv, b * BLKR + r)
                off = zp1 - rm * inv1
                k2 = rs * inv2

                def cb_body(cb):
                    o = cb * (U * L)
                    for u in range(U):
                        v = buf[r, pl.ds(o + u * L, L)]
                        e = plsc.load_gather(
                            lutv, [_code16(v, inv1, off, lane)])
                        t2m = e * k2 + zp2m
                        q2m = jnp.minimum(t2m, hi)
                        obuf[r, pl.ds(o + u * L, L)] = (q2m - zp2m) * s2

                plsc.parallel_loop(0, C // (U * L), 1, unroll=4)(cb_body)
                return 0

            lax.fori_loop(0, BLKR, row_body, 0)

        fill(0, xb0, si0)
        fill(1, xb1, si1)

        def outer(gi, _):
            b0 = gi * 2

            wait_fill(xb0, si0)

            @pl.when(b0 >= 2)
            def _():
                wait_drain(ob0, so0)

            process(b0, xb0, ob0)
            drain(b0, ob0, so0)

            @pl.when(b0 + 2 < nblk)
            def _():
                fill(b0 + 2, xb0, si0)

            wait_fill(xb1, si1)

            @pl.when(b0 >= 2)
            def _():
                wait_drain(ob1, so1)

            process(b0 + 1, xb1, ob1)
            drain(b0 + 1, ob1, so1)

            @pl.when(b0 + 3 < nblk)
            def _():
                fill(b0 + 3, xb1, si1)

            return 0

        lax.fori_loop(0, nblk // 2, outer, 0)
        wait_drain(ob0, so0)
        wait_drain(ob1, so1)

    return pass_c


def kernel(inputs):
    shape = inputs.shape
    NW = 32

    rowmax, gpart = _make_pass_a(shape, NW, L)(inputs)
    mn = jnp.min(gpart)
    mx = jnp.float32(0.0)
    scale1 = jnp.maximum((mx - mn) / 255.0, jnp.float32(1e-12))
    zp1 = jnp.clip(jnp.round(-128.0 - mn / scale1), -128, 127)
    codes = jnp.clip(jnp.arange(512, dtype=jnp.float32) - 256.0, 0.0, 255.0)
    lut = jnp.exp(((codes - 128.0) - zp1) * scale1)
    lut16 = jnp.repeat(lut, L)
    inv1 = 1.0 / scale1
    scal1 = jnp.concatenate(
        [jnp.full((L,), inv1), jnp.full((L,), zp1)]).astype(jnp.float32)

    invsum, pmn, pmx = _make_pass_b(shape, NW, L)(inputs, rowmax, lut16,
                                                  scal1)
    return _make_pass_c(shape, NW, L // 2)(inputs, rowmax, invsum, pmn, pmx,
                                           lut16, scal1)


# speedup vs baseline: 14.0711x; 1.0039x over previous
import functools

import jax
import jax.numpy as jnp
import numpy as np
from jax import lax
from jax.experimental import pallas as pl
from jax.experimental.pallas import tpu as pltpu
from jax.experimental.pallas import tpu_sc as plsc

_MAGIC = np.float32(12582912.0)
_KMAGIC = 1262485504
_KME = _KMAGIC - 384
L = 16
U = 4


def _mesh():
    return plsc.VectorSubcoreMesh(core_axis_name="c", subcore_axis_name="s")


def _wid():
    info = plsc.get_sparse_core_info()
    return lax.axis_index("s") * info.num_cores + lax.axis_index("c")


def _tree(fn, vs):
    vs = list(vs)
    while len(vs) > 1:
        vs = [fn(vs[i], vs[i + 1]) for i in range(0, len(vs) - 1, 2)] + (
            [vs[-1]] if len(vs) % 2 else [])
    return vs[0]


def _lane_iota():
    return lax.iota(jnp.int32, L)


def _code16(v, inv1, off, lane):
    t = v * inv1 + off
    iu = lax.bitcast_convert_type(t + _MAGIC, jnp.int32)
    return jnp.left_shift(iu - _KME, 4) + lane


def _splat(ref, i):
    return plsc.load_gather(ref, [jnp.full((L,), 0, jnp.int32) + i])


def _hs(row, sh, mask):
    s = jnp.bitwise_and(row, mask)
    return jnp.right_shift(row, sh), pl.multiple_of(s, 8)


def _make_pass_a(shape, NW, BLKR):
    _, H, S, C = shape
    R = H * S
    rows_w = R // NW
    nblk = rows_w // BLKR
    sh, mask = (S - 1).bit_length(), S - 1

    @functools.partial(
        pl.kernel,
        out_type=(
            jax.ShapeDtypeStruct((R,), jnp.float32),
            jax.ShapeDtypeStruct((NW * L,), jnp.float32),
        ),
        mesh=_mesh(),
        compiler_params=pltpu.CompilerParams(needs_layout_passes=False),
        scratch_types=[
            pltpu.VMEM((BLKR, C), jnp.float32),
            pltpu.VMEM((BLKR, C), jnp.float32),
            pltpu.VMEM((BLKR, C), jnp.float32),
            pltpu.VMEM((rows_w,), jnp.float32),
            pltpu.VMEM((L,), jnp.float32),
            pltpu.SemaphoreType.DMA,
            pltpu.SemaphoreType.DMA,
            pltpu.SemaphoreType.DMA,
        ],
    )
    def pass_a(x4, rowmax_hbm, gpart_hbm, xb0, xb1, xb2, rmv, gv,
               s0, s1, s2):
        wid = _wid()
        base = wid * rows_w
        lane = _lane_iota()

        def fill(b, buf, sem):
            h, s = _hs(base + b * BLKR, sh, mask)
            pltpu.async_copy(x4.at[0, h, pl.ds(s, BLKR), :], buf, sem)

        def wait_fill(buf, sem):
            pltpu.make_async_copy(
                x4.at[0, 0, pl.ds(0, BLKR), :], buf, sem).wait()

        def process(b, buf, g):
            def row_body(r, carry):
                g, rmblk = carry

                def cb_body(cb, acc):
                    mx, mn = acc
                    o = cb * (U * L)
                    vs = [buf[r, pl.ds(o + u * L, L)] for u in range(U)]
                    return (jnp.maximum(mx, _tree(jnp.maximum, vs)),
                            jnp.minimum(mn, _tree(jnp.minimum, vs)))

                mxv, mnv = plsc.parallel_loop(
                    0, C // (U * L), 1, unroll=4,
                    carry=(jnp.full((L,), -jnp.inf, jnp.float32),
                           jnp.full((L,), jnp.inf, jnp.float32)),
                )(cb_body)
                rmax = jnp.max(mxv)
                rmin = jnp.min(mnv)
                rmblk = jnp.where(lane == r, jnp.full((L,), rmax), rmblk)
                return (jnp.minimum(g, rmin - rmax), rmblk)

            g, rmblk = lax.fori_loop(
                0, BLKR, row_body, (g, jnp.zeros((L,), jnp.float32)))
            rmv[pl.ds(b * BLKR, L)] = rmblk
            return g

        fill(0, xb0, s0)
        fill(1, xb1, s1)
        fill(2, xb2, s2)

        def outer(gi, g):
            b0 = gi * 3
            for p, (buf, sem) in enumerate(
                    ((xb0, s0), (xb1, s1), (xb2, s2))):
                b = b0 + p
                wait_fill(buf, sem)
                g = process(b, buf, g)

                @pl.when(b + 3 < nblk)
                def _():
                    fill(b + 3, buf, sem)

            return g

        g = lax.fori_loop(0, nblk // 3, outer, jnp.float32(jnp.inf))
        gv[...] = jnp.full((L,), g)
        pltpu.sync_copy(rmv, rowmax_hbm.at[pl.ds(base, rows_w)])
        pltpu.sync_copy(gv, gpart_hbm.at[pl.ds(wid * L, L)])

    return pass_a


def _make_pass_b(shape, NW, BLKR):
    _, H, S, C = shape
    R = H * S
    rows_w = R // NW
    nblk = rows_w // BLKR
    sh, mask = (S - 1).bit_length(), S - 1

    @functools.partial(
        pl.kernel,
        out_type=(
            jax.ShapeDtypeStruct((R,), jnp.float32),
            jax.ShapeDtypeStruct((NW * L,), jnp.float32),
            jax.ShapeDtypeStruct((NW * L,), jnp.float32),
        ),
        mesh=_mesh(),
        compiler_params=pltpu.CompilerParams(needs_layout_passes=False),
        scratch_types=[
            pltpu.VMEM((BLKR, C), jnp.float32),
            pltpu.VMEM((BLKR, C), jnp.float32),
            pltpu.VMEM((rows_w,), jnp.float32),
            pltpu.VMEM((rows_w,), jnp.float32),
            pltpu.VMEM((rows_w,), jnp.float32),
            pltpu.VMEM((512 * L,), jnp.float32),
            pltpu.VMEM((2 * L,), jnp.float32),
            pltpu.SemaphoreType.DMA,
            pltpu.SemaphoreType.DMA,
        ],
    )
    def pass_b(x4, rowmax_hbm, lut_hbm, scal_hbm, invsum_hbm, pmn_hbm,
               pmx_hbm, xb0, xb1, rmv, rsv, emv, lutv, scv, s0, s1):
        wid = _wid()
        base = wid * rows_w
        lane = _lane_iota()
        pltpu.sync_copy(lut_hbm, lutv)
        pltpu.sync_copy(scal_hbm, scv)
        pltpu.sync_copy(rowmax_hbm.at[pl.ds(base, rows_w)], rmv)
        inv1 = scv[pl.ds(0, L)]
        zp1 = scv[pl.ds(L, L)]

        def fill(b, buf, sem):
            h, s = _hs(base + b * BLKR, sh, mask)
            pltpu.async_copy(x4.at[0, h, pl.ds(s, BLKR), :], buf, sem)

        def wait_fill(buf, sem):
            pltpu.make_async_copy(
                x4.at[0, 0, pl.ds(0, BLKR), :], buf, sem).wait()

        def process(b, buf):
            def row_body(r, carry):
                rsblk, emblk = carry
                rm = _splat(rmv, b * BLKR + r)
                off = zp1 - rm * inv1

                def cb_body(cb, acc):
                    sacc, macc = acc
                    o = cb * (U * L)
                    es = []
                    for u in range(U):
                        v = buf[r, pl.ds(o + u * L, L)]
                        es.append(
                            plsc.load_gather(lutv,
                                             [_code16(v, inv1, off, lane)]))
                    return (sacc + _tree(jnp.add, es),
                            jnp.minimum(macc, _tree(jnp.minimum, es)))

                sacc, macc = plsc.parallel_loop(
                    0, C // (U * L), 1, unroll=2,
                    carry=(jnp.zeros((L,), jnp.float32),
                           jnp.full((L,), jnp.inf, jnp.float32)),
                )(cb_body)
                rsum = jnp.sum(sacc)
                rmin = jnp.min(macc)
                rsblk = jnp.where(lane == r, jnp.full((L,), rsum), rsblk)
                emblk = jnp.where(lane == r, jnp.full((L,), rmin), emblk)
                return (rsblk, emblk)

            rsblk, emblk = lax.fori_loop(
                0, BLKR, row_body,
                (jnp.zeros((L,), jnp.float32), jnp.zeros((L,), jnp.float32)))
            rsv[pl.ds(b * BLKR, L)] = rsblk
            emv[pl.ds(b * BLKR, L)] = emblk

        fill(0, xb0, s0)
        fill(1, xb1, s1)

        def outer(gi, _):
            b0 = gi * 2

            wait_fill(xb0, s0)
            process(b0, xb0)

            @pl.when(b0 + 2 < nblk)
            def _():
                fill(b0 + 2, xb0, s0)

            wait_fill(xb1, s1)
            process(b0 + 1, xb1)

            @pl.when(b0 + 3 < nblk)
            def _():
                fill(b0 + 3, xb1, s1)

            return 0

        lax.fori_loop(0, nblk // 2, outer, 0)

        def ep_body(i, carry):
            omn, omx = carry
            rs = rsv[pl.ds(i * L, L)]
            em = emv[pl.ds(i * L, L)]
            inv = 1.0 / rs
            rsv[pl.ds(i * L, L)] = inv
            return (jnp.minimum(omn, em * inv), jnp.maximum(omx, inv))

        omn, omx = lax.fori_loop(
            0, rows_w // L, ep_body,
            (jnp.full((L,), jnp.inf, jnp.float32),
             jnp.full((L,), -jnp.inf, jnp.float32)))
        emv[pl.ds(0, L)] = omn
        emv[pl.ds(L, L)] = omx
        pltpu.sync_copy(rsv, invsum_hbm.at[pl.ds(base, rows_w)])
        pltpu.sync_copy(emv.at[pl.ds(0, L)], pmn_hbm.at[pl.ds(wid * L, L)])
        pltpu.sync_copy(emv.at[pl.ds(L, L)], pmx_hbm.at[pl.ds(wid * L, L)])

    return pass_b


def _make_pass_c(shape, NW, BLKR):
    _, H, S, C = shape
    R = H * S
    rows_w = R // NW
    nblk = rows_w // BLKR
    sh, mask = (S - 1).bit_length(), S - 1

    @functools.partial(
        pl.kernel,
        out_type=jax.ShapeDtypeStruct(shape, jnp.float32),
        mesh=_mesh(),
        compiler_params=pltpu.CompilerParams(needs_layout_passes=False),
        scratch_types=[
            pltpu.VMEM((BLKR, C), jnp.float32),
            pltpu.VMEM((BLKR, C), jnp.float32),
            pltpu.VMEM((BLKR, C), jnp.float32),
            pltpu.VMEM((BLKR, C), jnp.float32),
            pltpu.VMEM((rows_w,), jnp.float32),
            pltpu.VMEM((rows_w,), jnp.float32),
            pltpu.VMEM((512 * L,), jnp.float32),
            pltpu.VMEM((2 * L,), jnp.float32),
            pltpu.VMEM((NW * L,), jnp.float32),
            pltpu.VMEM((NW * L,), jnp.float32),
            pltpu.SemaphoreType.DMA,
            pltpu.SemaphoreType.DMA,
            pltpu.SemaphoreType.DMA,
            pltpu.SemaphoreType.DMA,
        ],
    )
    def pass_c(x4, rowmax_hbm, invsum_hbm, pmn_hbm, pmx_hbm, lut_hbm,
               scal_hbm, out4, xb0, xb1, ob0, ob1, rmv, rsv, lutv, scv,
               pmnv, pmxv, si0, si1, so0, so1):
        wid = _wid()
        base = wid * rows_w
        lane = _lane_iota()
        pltpu.sync_copy(lut_hbm, lutv)
        pltpu.sync_copy(scal_hbm, scv)
        pltpu.sync_copy(rowmax_hbm.at[pl.ds(base, rows_w)], rmv)
        pltpu.sync_copy(invsum_hbm.at[pl.ds(base, rows_w)], rsv)
        pltpu.sync_copy(pmn_hbm, pmnv)
        pltpu.sync_copy(pmx_hbm, pmxv)
        inv1 = scv[pl.ds(0, L)]
        zp1 = scv[pl.ds(L, L)]
        hi = np.float32(12582912.0 + 127.0)

        def red_body(i, carry):
            a, b2 = carry
            return (jnp.minimum(a, pmnv[pl.ds(i * L, L)]),
                    jnp.maximum(b2, pmxv[pl.ds(i * L, L)]))

        omnv, omxv = lax.fori_loop(
            0, NW, red_body,
            (jnp.full((L,), jnp.inf, jnp.float32),
             jnp.full((L,), -jnp.inf, jnp.float32)))
        omn = jnp.full((L,), jnp.min(omnv))
        omx = jnp.full((L,), jnp.max(omxv))
        s2 = jnp.maximum((omx - omn) / 255.0, jnp.float32(1e-12))
        zp2 = jnp.clip(((-128.0 - omn / s2) + _MAGIC) - _MAGIC,
                       -128.0, 127.0)
        inv2 = 1.0 / s2
        zp2m = zp2 + _MAGIC

        def fill(b, buf, sem):
            h, s = _hs(base + b * BLKR, sh, mask)
            pltpu.async_copy(x4.at[0, h, pl.ds(s, BLKR), :], buf, sem)

        def wait_fill(buf, sem):
            pltpu.make_async_copy(
                x4.at[0, 0, pl.ds(0, BLKR), :], buf, sem).wait()

        def drain(b, buf, sem):
            h, s = _hs(base + b * BLKR, sh, mask)
            pltpu.async_copy(buf, out4.at[0, h, pl.ds(s, BLKR), :], sem)

        def wait_drain(buf, sem):
            pltpu.make_async_copy(
                buf, out4.at[0, 0, pl.ds(0, BLKR), :], sem).wait()

        def process(b, buf, obuf):
            def row_body(r, _):
                rm = _splat(rmv, b * BLKR + r)
                rs = _splat(rsv, b * BLKR + r)
                off = zp1 - rm * inv1
                k2 = rs * inv2

                def cb_body(cb):
                    o = cb * (U * L)
                    for u in range(U):
                        v = buf[r, pl.ds(o + u * L, L)]
                        e = plsc.load_gather(
                            lutv, [_code16(v, inv1, off, lane)])
                        t2m = e * k2 + zp2m
                        q2m = jnp.minimum(t2m, hi)
                        obuf[r, pl.ds(o + u * L, L)] = (q2m - zp2m) * s2

                plsc.parallel_loop(0, C // (U * L), 1, unroll=2)(cb_body)
                return 0

            lax.fori_loop(0, BLKR, row_body, 0)

        fill(0, xb0, si0)
        fill(1, xb1, si1)

        def outer(gi, _):
            b0 = gi * 2

            wait_fill(xb0, si0)

            @pl.when(b0 >= 2)
            def _():
                wait_drain(ob0, so0)

            process(b0, xb0, ob0)
            drain(b0, ob0, so0)

            @pl.when(b0 + 2 < nblk)
            def _():
                fill(b0 + 2, xb0, si0)

            wait_fill(xb1, si1)

            @pl.when(b0 >= 2)
            def _():
                wait_drain(ob1, so1)

            process(b0 + 1, xb1, ob1)
            drain(b0 + 1, ob1, so1)

            @pl.when(b0 + 3 < nblk)
            def _():
                fill(b0 + 3, xb1, si1)

            return 0

        lax.fori_loop(0, nblk // 2, outer, 0)
        wait_drain(ob0, so0)
        wait_drain(ob1, so1)

    return pass_c


def kernel(inputs):
    shape = inputs.shape
    NW = 32

    rowmax, gpart = _make_pass_a(shape, NW, L)(inputs)
    mn = jnp.min(gpart)
    mx = jnp.float32(0.0)
    scale1 = jnp.maximum((mx - mn) / 255.0, jnp.float32(1e-12))
    zp1 = jnp.clip(jnp.round(-128.0 - mn / scale1), -128, 127)
    codes = jnp.clip(jnp.arange(512, dtype=jnp.float32) - 256.0, 0.0, 255.0)
    lut = jnp.exp(((codes - 128.0) - zp1) * scale1)
    lut16 = jnp.repeat(lut, L)
    inv1 = 1.0 / scale1
    scal1 = jnp.concatenate(
        [jnp.full((L,), inv1), jnp.full((L,), zp1)]).astype(jnp.float32)

    invsum, pmn, pmx = _make_pass_b(shape, NW, L)(inputs, rowmax, lut16,
                                                  scal1)
    return _make_pass_c(shape, NW, L // 2)(inputs, rowmax, invsum, pmn, pmx,
                                           lut16, scal1)
